# baseline (speedup 1.0000x reference)
import jax
import jax.numpy as jnp
from jax.experimental import pallas as pl

_VARIANCES = jnp.array([0.1, 0.1, 0.2, 0.2], dtype=jnp.float32)
_L = 91
_MAX_TOTAL = 200
_SCORE_THR = 0.5
_IOU_THR = 0.5
_CAND = 256


def _decode(rois, deltas):
    h = rois[..., 2] - rois[..., 0]
    w = rois[..., 3] - rois[..., 1]
    cy = rois[..., 0] + 0.5 * h
    cx = rois[..., 1] + 0.5 * w
    dy, dx, dh, dw = deltas[..., 0], deltas[..., 1], deltas[..., 2], deltas[..., 3]
    ph = jnp.exp(dh) * h
    pw = jnp.exp(dw) * w
    pcy = dy * h + cy
    pcx = dx * w + cx
    y1 = pcy - 0.5 * ph
    x1 = pcx - 0.5 * pw
    return jnp.stack([y1, x1, y1 + ph, x1 + pw], axis=-1)


def _iou(b):
    y1, x1, y2, x2 = b[:, 0], b[:, 1], b[:, 2], b[:, 3]
    area = jnp.maximum(y2 - y1, 0.0) * jnp.maximum(x2 - x1, 0.0)
    iy1 = jnp.maximum(y1[:, None], y1[None, :])
    ix1 = jnp.maximum(x1[:, None], x1[None, :])
    iy2 = jnp.minimum(y2[:, None], y2[None, :])
    ix2 = jnp.minimum(x2[:, None], x2[None, :])
    inter = jnp.maximum(iy2 - iy1, 0.0) * jnp.maximum(ix2 - ix1, 0.0)
    union = area[:, None] + area[None, :] - inter
    return inter / jnp.maximum(union, 1e-8)


def _nms1(boxes, scores):
    s, idx = jax.lax.top_k(scores, _CAND)
    b = boxes[idx]
    iou = _iou(b)
    valid = s > _SCORE_THR
    pos = jnp.arange(_CAND)

    def body(i, keep):
        sup = (iou[i] > _IOU_THR) & (pos > i) & keep[i]
        return keep & (~sup)

    keep = jax.lax.fori_loop(0, _CAND, body, valid)
    rank = jnp.cumsum(keep.astype(jnp.int32)) - 1
    keep = keep & (rank < _MAX_TOTAL)
    out_s = jnp.where(keep, s, -1.0)
    return b, out_s


def kernel(roi_bboxes, pred_deltas, pred_label_probs):
    B = roi_bboxes.shape[0]
    deltas = pred_deltas.reshape(B, -1, _L, 4) * _VARIANCES
    expanded_rois = jnp.tile(roi_bboxes[:, :, None, :], (1, 1, _L, 1))
    pred_bboxes = _decode(expanded_rois, deltas)
    labels_map = jnp.argmax(pred_label_probs, axis=-1)[..., None]
    scores = jnp.where(labels_map != 0, pred_label_probs, jnp.zeros_like(pred_label_probs))
    boxes_t = jnp.transpose(pred_bboxes, (0, 2, 1, 3))
    scores_t = jnp.transpose(scores, (0, 2, 1))
    nb, ns = jax.vmap(jax.vmap(_nms1))(boxes_t, scores_t)
    flat_s = ns.reshape(B, -1)
    flat_b = nb.reshape(B, -1, 4)
    top_s, top_i = jax.lax.top_k(flat_s, _MAX_TOTAL)
    final_bboxes = jnp.take_along_axis(flat_b, top_i[..., None], axis=1)
    labels = (top_i // _CAND).astype(jnp.float32)
    valid = top_s > 0.0
    final_scores = jnp.where(valid, top_s, 0.0)
    final_bboxes = jnp.where(valid[..., None], final_bboxes, 0.0)
    final_labels = jnp.where(valid, labels, 0.0)
    return final_bboxes, final_labels, final_scores


# speedup vs baseline: 2.7095x; 2.7095x over previous
import functools

import numpy as np
import jax
import jax.numpy as jnp
from jax import lax
from jax.experimental import pallas as pl
from jax.experimental.pallas import tpu as pltpu

_L = 91
_LP = 96
_N = 20000
_CAND = 256
_MAX_TOTAL = 200
_SCORE_THR = 0.5
_IOU_THR = 0.5
_CHUNK = 512
_T = 64
_INT_MIN = np.int32(-2**31)
_IDX_PAD = np.int32(2**31 - 1)


def _f2key(s):
    b = lax.bitcast_convert_type(s, jnp.int32)
    return b ^ (lax.shift_right_arithmetic(b, 31) & np.int32(0x7FFFFFFF))


def _key2f(k):
    return lax.bitcast_convert_type(
        k ^ (lax.shift_right_arithmetic(k, 31) & np.int32(0x7FFFFFFF)), jnp.float32)


def _gt(ka, ia, kb, ib):
    return (ka > kb) | ((ka == kb) & (ia < ib))


def _stage(karr, iarr, T, S, j, k, invert=False):
    if j < T:
        G = T // (2 * j)
        k5 = karr.reshape(G, 2, j, S, 128)
        i5 = iarr.reshape(G, 2, j, S, 128)
        ka, kb = k5[:, 0], k5[:, 1]
        ia, ib = i5[:, 0], i5[:, 1]
        gtba = _gt(kb, ib, ka, ia)
        if k is None:
            swap = gtba
        else:
            sh = (G, j, S, 128)
            n_a = (lax.broadcasted_iota(jnp.int32, sh, 2) * T
                   + lax.broadcasted_iota(jnp.int32, sh, 0) * (2 * j)
                   + lax.broadcasted_iota(jnp.int32, sh, 1))
            dir_a = ((n_a & k) != 0) if invert else ((n_a & k) == 0)
            swap = gtba == dir_a
        nak = jnp.where(swap, kb, ka)
        nbk = jnp.where(swap, ka, kb)
        nai = jnp.where(swap, ib, ia)
        nbi = jnp.where(swap, ia, ib)
        karr = jnp.concatenate([nak[:, None], nbk[:, None]], axis=1).reshape(T, S, 128)
        iarr = jnp.concatenate([nai[:, None], nbi[:, None]], axis=1).reshape(T, S, 128)
    else:
        js = j // T
        sh = (T, S, 128)
        si = lax.broadcasted_iota(jnp.int32, sh, 1)
        low = (si & js) == 0
        pk = jnp.where(low, jnp.roll(karr, -js, axis=1), jnp.roll(karr, js, axis=1))
        pi = jnp.where(low, jnp.roll(iarr, -js, axis=1), jnp.roll(iarr, js, axis=1))
        gtp = _gt(pk, pi, karr, iarr)
        if k is None:
            winner_here = low
        else:
            n = si * T + lax.broadcasted_iota(jnp.int32, sh, 0)
            d = ((n & k) != 0) if invert else ((n & k) == 0)
            winner_here = low == d
        take = gtp == winner_here
        karr = jnp.where(take, pk, karr)
        iarr = jnp.where(take, pi, iarr)
    return karr, iarr


def _topk_body(in_ref, out_s_ref, out_i_ref, rk_ref, ri_ref, *,
               n_valid, n_chunks, compute_scores):
    c = pl.program_id(1)

    x = in_ref[0].reshape(_T, 8, 128)
    lane = lax.broadcasted_iota(jnp.int32, (_T, 8, 128), 2)
    if compute_scores:
        p = jnp.where(lane < _L, x, -1.0)
        amax = jnp.max(p, axis=2, keepdims=True)
        argl = jnp.min(jnp.where(p == amax, lane, 128), axis=2, keepdims=True)
        s = jnp.where(argl == 0, 0.0, p)
    else:
        s = x
    key = _f2key(s)
    t_i = lax.broadcasted_iota(jnp.int32, (_T, 8, 128), 0)
    s_i = lax.broadcasted_iota(jnp.int32, (_T, 8, 128), 1)
    rowid = c * _CHUNK + t_i * 8 + s_i
    key = jnp.where(rowid < n_valid, key, _INT_MIN)

    for k in [2, 4, 8, 16, 32, 64, 128, 256, 512]:
        j = k // 2
        while j >= 1:
            key, rowid = _stage(key, rowid, _T, 8, j, k, invert=True)
            j //= 2

    akr, air = key[:, 4:8], rowid[:, 4:8]

    @pl.when(c == 0)
    def _init():
        rk_ref[...] = jnp.full((_T, 4, 128), _INT_MIN, jnp.int32)
        ri_ref[...] = jnp.full((_T, 4, 128), _IDX_PAD, jnp.int32)

    rk, ri = rk_ref[...], ri_ref[...]
    gta = _gt(akr, air, rk, ri)
    wk = jnp.where(gta, akr, rk)
    wi = jnp.where(gta, air, ri)
    for j in [128, 64, 32, 16, 8, 4, 2, 1]:
        wk, wi = _stage(wk, wi, _T, 4, j, None)
    rk_ref[...] = wk
    ri_ref[...] = wi

    @pl.when(c == n_chunks - 1)
    def _emit():
        sc = _key2f(wk)
        for si in range(4):
            out_s_ref[0, si] = sc[:, si, :]
            out_i_ref[0, si] = wi[:, si, :]


def _topk256(x, n_valid, compute_scores):
    B, n, _ = x.shape
    n_chunks = (n + _CHUNK - 1) // _CHUNK
    out = pl.pallas_call(
        functools.partial(_topk_body, n_valid=n_valid, n_chunks=n_chunks,
                          compute_scores=compute_scores),
        grid=(B, n_chunks),
        in_specs=[pl.BlockSpec((1, _CHUNK, 128), lambda b, c: (b, c, 0))],
        out_specs=[pl.BlockSpec((1, 4, _T, 128), lambda b, c: (b, 0, 0, 0)),
                   pl.BlockSpec((1, 4, _T, 128), lambda b, c: (b, 0, 0, 0))],
        out_shape=[jax.ShapeDtypeStruct((B, 4, _T, 128), jnp.float32),
                   jax.ShapeDtypeStruct((B, 4, _T, 128), jnp.int32)],
        scratch_shapes=[pltpu.VMEM((_T, 4, 128), jnp.int32),
                        pltpu.VMEM((_T, 4, 128), jnp.int32)],
    )(x)
    return out[0].reshape(B, 256, 128), out[1].reshape(B, 256, 128)


_LB = 16


def _decode(r, d):
    h = r[2] - r[0]
    w = r[3] - r[1]
    cy = r[0] + 0.5 * h
    cx = r[1] + 0.5 * w
    dy = d[0] * np.float32(0.1)
    dx = d[1] * np.float32(0.1)
    dh = d[2] * np.float32(0.2)
    dw = d[3] * np.float32(0.2)
    ph = jnp.exp(dh) * h
    pw = jnp.exp(dw) * w
    pcy = dy * h + cy
    pcx = dx * w + cx
    y1 = pcy - 0.5 * ph
    x1 = pcx - 0.5 * pw
    y2 = y1 + ph
    x2 = x1 + pw
    return y1, x1, y2, x2


def _nms_body(s_ref, r_ref, d_ref, out_s_ref, out_b_ref, iou_ref):
    y1, x1, y2, x2 = _decode(r_ref[0], d_ref[0])
    area = jnp.maximum(y2 - y1, 0.0) * jnp.maximum(x2 - x1, 0.0)

    iy1 = jnp.maximum(y1[:, :, None], y1[:, None, :])
    ix1 = jnp.maximum(x1[:, :, None], x1[:, None, :])
    iy2 = jnp.minimum(y2[:, :, None], y2[:, None, :])
    ix2 = jnp.minimum(x2[:, :, None], x2[:, None, :])
    inter = jnp.maximum(iy2 - iy1, 0.0) * jnp.maximum(ix2 - ix1, 0.0)
    union = area[:, :, None] + area[:, None, :] - inter
    iou_ref[...] = inter / jnp.maximum(union, 1e-8)

    s = s_ref[0]
    jlane = lax.broadcasted_iota(jnp.int32, (_LB, 256), 1)

    def body(i, keep):
        row = iou_ref[:, pl.ds(i, 1), :].reshape(_LB, 256)
        keep_i = jnp.sum(jnp.where(jlane == i, keep, 0.0),
                         axis=1, keepdims=True)
        sup = (row > _IOU_THR) & (jlane > i) & (keep_i > 0.0)
        return jnp.where(sup, 0.0, keep)

    keep_f = lax.fori_loop(0, _CAND, body,
                           jnp.where(s > _SCORE_THR, 1.0, 0.0))
    keep = keep_f > 0.0
    lt = (lax.broadcasted_iota(jnp.int32, (256, 256), 0)
          <= lax.broadcasted_iota(jnp.int32, (256, 256), 1)).astype(jnp.float32)
    cum = jnp.dot(keep_f, lt, preferred_element_type=jnp.float32)
    keep = keep & (cum - 1.0 < np.float32(_MAX_TOTAL))
    out_s_ref[0] = jnp.where(keep, s, -1.0)
    out_b_ref[0, 0] = y1
    out_b_ref[0, 1] = x1
    out_b_ref[0, 2] = y2
    out_b_ref[0, 3] = x2


def _nms(s_in, rois_c, deltas_c):
    B = s_in.shape[0]
    return pl.pallas_call(
        _nms_body,
        grid=(B, _LP // _LB),
        in_specs=[
            pl.BlockSpec((1, _LB, 256), lambda b, l: (b, l, 0)),
            pl.BlockSpec((1, 4, _LB, 256), lambda b, l: (b, 0, l, 0)),
            pl.BlockSpec((1, 4, _LB, 256), lambda b, l: (b, 0, l, 0)),
        ],
        out_specs=[pl.BlockSpec((1, _LB, 256), lambda b, l: (b, l, 0)),
                   pl.BlockSpec((1, 4, _LB, 256), lambda b, l: (b, 0, l, 0))],
        out_shape=[jax.ShapeDtypeStruct((B, _LP, 256), jnp.float32),
                   jax.ShapeDtypeStruct((B, 4, _LP, 256), jnp.float32)],
        scratch_shapes=[pltpu.VMEM((_LB, 256, 256), jnp.float32)],
    )(s_in, rois_c, deltas_c)


def kernel(roi_bboxes, pred_deltas, pred_label_probs):
    B = roi_bboxes.shape[0]

    s1, i1 = _topk256(pred_label_probs, _N, compute_scores=True)
    s1 = jnp.transpose(s1[:, :, :_L], (0, 2, 1))
    i1 = jnp.transpose(i1[:, :, :_L], (0, 2, 1))

    s_in = jnp.concatenate(
        [s1, jnp.full((B, _LP - _L, 256), -2.0, jnp.float32)], axis=1)
    idx = jnp.concatenate(
        [i1, jnp.zeros((B, _LP - _L, 256), jnp.int32)], axis=1)
    idx = jnp.clip(idx, 0, _N - 1)

    flat_idx = idx.reshape(B, -1)
    rois_g = jax.vmap(lambda r, i: r[i])(roi_bboxes, flat_idx)
    lcls = jnp.arange(_LP, dtype=jnp.int32)[None, :, None]
    drow = jnp.clip(idx * _L + jnp.minimum(lcls, _L - 1), 0, _N * _L - 1)
    dflat = pred_deltas.reshape(B, _N * _L, 4)
    deltas_g = jax.vmap(lambda d, i: d[i])(dflat, drow.reshape(B, -1))

    rois_c = jnp.transpose(rois_g.reshape(B, _LP, 256, 4), (0, 3, 1, 2))
    deltas_c = jnp.transpose(deltas_g.reshape(B, _LP, 256, 4), (0, 3, 1, 2))

    out_s, out_b = _nms(s_in, rois_c, deltas_c)

    flat_s = out_s.reshape(B, _LP * 256)
    s2, i2 = _topk256(jnp.transpose(flat_s)[None], _LP * 256,
                      compute_scores=False)
    top_s = jnp.transpose(s2[0, :_MAX_TOTAL, :B])
    top_i = jnp.transpose(i2[0, :_MAX_TOTAL, :B])

    flat_b = jnp.transpose(out_b, (0, 2, 3, 1)).reshape(B, _LP * 256, 4)
    final_bboxes = jnp.take_along_axis(flat_b, top_i[..., None], axis=1)
    labels = (top_i // _CAND).astype(jnp.float32)
    valid = top_s > 0.0
    final_scores = jnp.where(valid, top_s, 0.0)
    final_bboxes = jnp.where(valid[..., None], final_bboxes, 0.0)
    final_labels = jnp.where(valid, labels, 0.0)
    return final_bboxes, final_labels, final_scores


# speedup vs baseline: 5.1221x; 1.8905x over previous
import functools

import numpy as np
import jax
import jax.numpy as jnp
from jax import lax
from jax.experimental import pallas as pl
from jax.experimental.pallas import tpu as pltpu

_L = 91
_LP = 96
_N = 20000
_CAND = 256
_MAX_TOTAL = 200
_SCORE_THR = 0.5
_IOU_THR = 0.5
_CHUNK = 512
_T = 64
_INT_MIN = np.int32(-2**31)
_IDX_PAD = np.int32(2**31 - 1)


def _f2key(s):
    b = lax.bitcast_convert_type(s, jnp.int32)
    return b ^ (lax.shift_right_arithmetic(b, 31) & np.int32(0x7FFFFFFF))


def _key2f(k):
    return lax.bitcast_convert_type(
        k ^ (lax.shift_right_arithmetic(k, 31) & np.int32(0x7FFFFFFF)), jnp.float32)


def _gt(ka, ia, kb, ib):
    return (ka > kb) | ((ka == kb) & (ia < ib))


def _stage(karr, iarr, T, S, j, k, invert=False):
    if j < T:
        G = T // (2 * j)
        k5 = karr.reshape(G, 2, j, S, 128)
        i5 = iarr.reshape(G, 2, j, S, 128)
        ka, kb = k5[:, 0], k5[:, 1]
        ia, ib = i5[:, 0], i5[:, 1]
        gtba = _gt(kb, ib, ka, ia)
        if k is None:
            swap = gtba
        else:
            sh = (G, j, S, 128)
            n_a = (lax.broadcasted_iota(jnp.int32, sh, 2) * T
                   + lax.broadcasted_iota(jnp.int32, sh, 0) * (2 * j)
                   + lax.broadcasted_iota(jnp.int32, sh, 1))
            dir_a = ((n_a & k) != 0) if invert else ((n_a & k) == 0)
            swap = gtba == dir_a
        nak = jnp.where(swap, kb, ka)
        nbk = jnp.where(swap, ka, kb)
        nai = jnp.where(swap, ib, ia)
        nbi = jnp.where(swap, ia, ib)
        karr = jnp.concatenate([nak[:, None], nbk[:, None]], axis=1).reshape(T, S, 128)
        iarr = jnp.concatenate([nai[:, None], nbi[:, None]], axis=1).reshape(T, S, 128)
    else:
        js = j // T
        sh = (T, S, 128)
        si = lax.broadcasted_iota(jnp.int32, sh, 1)
        low = (si & js) == 0
        pk = jnp.where(low, jnp.roll(karr, -js, axis=1), jnp.roll(karr, js, axis=1))
        pi = jnp.where(low, jnp.roll(iarr, -js, axis=1), jnp.roll(iarr, js, axis=1))
        gtp = _gt(pk, pi, karr, iarr)
        if k is None:
            winner_here = low
        else:
            n = si * T + lax.broadcasted_iota(jnp.int32, sh, 0)
            d = ((n & k) != 0) if invert else ((n & k) == 0)
            winner_here = low == d
        take = gtp == winner_here
        karr = jnp.where(take, pk, karr)
        iarr = jnp.where(take, pi, iarr)
    return karr, iarr


def _topk_body(in_ref, out_s_ref, out_i_ref, rk_ref, ri_ref, *,
               n_valid, n_chunks, compute_scores):
    c = pl.program_id(1)

    x = in_ref[0].reshape(_T, 8, 128)
    lane = lax.broadcasted_iota(jnp.int32, (_T, 8, 128), 2)
    if compute_scores:
        p = jnp.where(lane < _L, x, -1.0)
        amax = jnp.max(p, axis=2, keepdims=True)
        argl = jnp.min(jnp.where(p == amax, lane, 128), axis=2, keepdims=True)
        s = jnp.where(argl == 0, 0.0, p)
    else:
        s = x
    key = _f2key(s)
    t_i = lax.broadcasted_iota(jnp.int32, (_T, 8, 128), 0)
    s_i = lax.broadcasted_iota(jnp.int32, (_T, 8, 128), 1)
    rowid = c * _CHUNK + t_i * 8 + s_i
    key = jnp.where(rowid < n_valid, key, _INT_MIN)

    for k in [2, 4, 8, 16, 32, 64, 128, 256, 512]:
        j = k // 2
        while j >= 1:
            key, rowid = _stage(key, rowid, _T, 8, j, k, invert=True)
            j //= 2

    akr, air = key[:, 4:8], rowid[:, 4:8]

    @pl.when(c == 0)
    def _init():
        rk_ref[...] = jnp.full((_T, 4, 128), _INT_MIN, jnp.int32)
        ri_ref[...] = jnp.full((_T, 4, 128), _IDX_PAD, jnp.int32)

    rk, ri = rk_ref[...], ri_ref[...]
    gta = _gt(akr, air, rk, ri)
    wk = jnp.where(gta, akr, rk)
    wi = jnp.where(gta, air, ri)
    for j in [128, 64, 32, 16, 8, 4, 2, 1]:
        wk, wi = _stage(wk, wi, _T, 4, j, None)
    rk_ref[...] = wk
    ri_ref[...] = wi

    @pl.when(c == n_chunks - 1)
    def _emit():
        sc = _key2f(wk)
        for si in range(4):
            out_s_ref[0, si] = sc[:, si, :]
            out_i_ref[0, si] = wi[:, si, :]


def _topk256(x, n_valid, compute_scores):
    B, n, _ = x.shape
    n_chunks = (n + _CHUNK - 1) // _CHUNK
    out = pl.pallas_call(
        functools.partial(_topk_body, n_valid=n_valid, n_chunks=n_chunks,
                          compute_scores=compute_scores),
        grid=(B, n_chunks),
        in_specs=[pl.BlockSpec((1, _CHUNK, 128), lambda b, c: (b, c, 0))],
        out_specs=[pl.BlockSpec((1, 4, _T, 128), lambda b, c: (b, 0, 0, 0)),
                   pl.BlockSpec((1, 4, _T, 128), lambda b, c: (b, 0, 0, 0))],
        out_shape=[jax.ShapeDtypeStruct((B, 4, _T, 128), jnp.float32),
                   jax.ShapeDtypeStruct((B, 4, _T, 128), jnp.int32)],
        scratch_shapes=[pltpu.VMEM((_T, 4, 128), jnp.int32),
                        pltpu.VMEM((_T, 4, 128), jnp.int32)],
    )(x)
    return out[0].reshape(B, 256, 128), out[1].reshape(B, 256, 128)


_LB = 16


def _decode(r, d):
    h = r[2] - r[0]
    w = r[3] - r[1]
    cy = r[0] + 0.5 * h
    cx = r[1] + 0.5 * w
    dy = d[0] * np.float32(0.1)
    dx = d[1] * np.float32(0.1)
    dh = d[2] * np.float32(0.2)
    dw = d[3] * np.float32(0.2)
    ph = jnp.exp(dh) * h
    pw = jnp.exp(dw) * w
    pcy = dy * h + cy
    pcx = dx * w + cx
    y1 = pcy - 0.5 * ph
    x1 = pcx - 0.5 * pw
    y2 = y1 + ph
    x2 = x1 + pw
    return y1, x1, y2, x2


def _nms_body(s_ref, r_ref, d_ref, out_s_ref, out_b_ref, iou_ref):
    y1, x1, y2, x2 = _decode(r_ref[0], d_ref[0])
    area = jnp.maximum(y2 - y1, 0.0) * jnp.maximum(x2 - x1, 0.0)

    iy1 = jnp.maximum(y1[:, :, None], y1[:, None, :])
    ix1 = jnp.maximum(x1[:, :, None], x1[:, None, :])
    iy2 = jnp.minimum(y2[:, :, None], y2[:, None, :])
    ix2 = jnp.minimum(x2[:, :, None], x2[:, None, :])
    inter = jnp.maximum(iy2 - iy1, 0.0) * jnp.maximum(ix2 - ix1, 0.0)
    union = area[:, :, None] + area[:, None, :] - inter
    iou_ref[...] = inter / jnp.maximum(union, 1e-8)

    s = s_ref[0]
    jlane = lax.broadcasted_iota(jnp.int32, (_LB, 256), 1)

    def body(i, keep):
        row = iou_ref[:, pl.ds(i, 1), :].reshape(_LB, 256)
        keep_i = jnp.sum(jnp.where(jlane == i, keep, 0.0),
                         axis=1, keepdims=True)
        sup = (row > _IOU_THR) & (jlane > i) & (keep_i > 0.0)
        return jnp.where(sup, 0.0, keep)

    keep_f = lax.fori_loop(0, _CAND, body,
                           jnp.where(s > _SCORE_THR, 1.0, 0.0))
    keep = keep_f > 0.0
    lt = (lax.broadcasted_iota(jnp.int32, (256, 256), 0)
          <= lax.broadcasted_iota(jnp.int32, (256, 256), 1)).astype(jnp.float32)
    cum = jnp.dot(keep_f, lt, preferred_element_type=jnp.float32)
    keep = keep & (cum - 1.0 < np.float32(_MAX_TOTAL))
    out_s_ref[0] = jnp.where(keep, s, -1.0)
    out_b_ref[0, 0] = y1
    out_b_ref[0, 1] = x1
    out_b_ref[0, 2] = y2
    out_b_ref[0, 3] = x2


def _nms(s_in, rois_c, deltas_c):
    B = s_in.shape[0]
    return pl.pallas_call(
        _nms_body,
        grid=(B, _LP // _LB),
        in_specs=[
            pl.BlockSpec((1, _LB, 256), lambda b, l: (b, l, 0)),
            pl.BlockSpec((1, 4, _LB, 256), lambda b, l: (b, 0, l, 0)),
            pl.BlockSpec((1, 4, _LB, 256), lambda b, l: (b, 0, l, 0)),
        ],
        out_specs=[pl.BlockSpec((1, _LB, 256), lambda b, l: (b, l, 0)),
                   pl.BlockSpec((1, 4, _LB, 256), lambda b, l: (b, 0, l, 0))],
        out_shape=[jax.ShapeDtypeStruct((B, _LP, 256), jnp.float32),
                   jax.ShapeDtypeStruct((B, 4, _LP, 256), jnp.float32)],
        scratch_shapes=[pltpu.VMEM((_LB, 256, 256), jnp.float32)],
    )(s_in, rois_c, deltas_c)


def kernel(roi_bboxes, pred_deltas, pred_label_probs):
    B = roi_bboxes.shape[0]

    s1, i1 = _topk256(pred_label_probs, _N, compute_scores=True)
    s1 = jnp.transpose(s1[:, :, :_L], (0, 2, 1))
    i1 = jnp.transpose(i1[:, :, :_L], (0, 2, 1))

    s_in = jnp.concatenate(
        [s1, jnp.full((B, _LP - _L, 256), -2.0, jnp.float32)], axis=1)
    idx = jnp.concatenate(
        [i1, jnp.zeros((B, _LP - _L, 256), jnp.int32)], axis=1)
    idx = jnp.clip(idx, 0, _N - 1)

    flat_idx = idx.reshape(B, -1)
    rois_g = jax.vmap(lambda r, i: r[i])(roi_bboxes, flat_idx)
    d_rows = jnp.take_along_axis(pred_deltas, flat_idx[..., None], axis=1)
    d_rows = d_rows.reshape(B, _LP, 256, _L * 4)
    lcls = jnp.minimum(jnp.arange(_LP, dtype=jnp.int32), _L - 1)
    cols = (4 * lcls[None, :, None, None]
            + jnp.arange(4, dtype=jnp.int32)[None, None, None, :])
    deltas_g = jnp.take_along_axis(
        d_rows, jnp.broadcast_to(cols, (B, _LP, 256, 4)), axis=3)

    rois_c = jnp.transpose(rois_g.reshape(B, _LP, 256, 4), (0, 3, 1, 2))
    deltas_c = jnp.transpose(deltas_g, (0, 3, 1, 2))

    out_s, out_b = _nms(s_in, rois_c, deltas_c)

    flat_s = out_s.reshape(B, _LP * 256)
    s2, i2 = _topk256(jnp.transpose(flat_s)[None], _LP * 256,
                      compute_scores=False)
    top_s = jnp.transpose(s2[0, :_MAX_TOTAL, :B])
    top_i = jnp.transpose(i2[0, :_MAX_TOTAL, :B])

    flat_b = jnp.transpose(out_b, (0, 2, 3, 1)).reshape(B, _LP * 256, 4)
    final_bboxes = jnp.take_along_axis(flat_b, top_i[..., None], axis=1)
    labels = (top_i // _CAND).astype(jnp.float32)
    valid = top_s > 0.0
    final_scores = jnp.where(valid, top_s, 0.0)
    final_bboxes = jnp.where(valid[..., None], final_bboxes, 0.0)
    final_labels = jnp.where(valid, labels, 0.0)
    return final_bboxes, final_labels, final_scores


# speedup vs baseline: 5.6986x; 1.1126x over previous
import functools

import numpy as np
import jax
import jax.numpy as jnp
from jax import lax
from jax.experimental import pallas as pl
from jax.experimental.pallas import tpu as pltpu

_L = 91
_LP = 96
_N = 20000
_CAND = 256
_MAX_TOTAL = 200
_SCORE_THR = 0.5
_IOU_THR = 0.5
_CHUNK = 512
_T = 64
_INT_MIN = np.int32(-2**31)
_IDX_PAD = np.int32(2**31 - 1)


def _f2key(s):
    b = lax.bitcast_convert_type(s, jnp.int32)
    return b ^ (lax.shift_right_arithmetic(b, 31) & np.int32(0x7FFFFFFF))


def _key2f(k):
    return lax.bitcast_convert_type(
        k ^ (lax.shift_right_arithmetic(k, 31) & np.int32(0x7FFFFFFF)), jnp.float32)


def _gt(ka, ia, kb, ib):
    return (ka > kb) | ((ka == kb) & (ia < ib))


def _stage(karr, iarr, T, S, j, k, invert=False, lane_level=None):
    if j < T:
        G = T // (2 * j)
        k5 = karr.reshape(G, 2, j, S, 128)
        i5 = iarr.reshape(G, 2, j, S, 128)
        ka, kb = k5[:, 0], k5[:, 1]
        ia, ib = i5[:, 0], i5[:, 1]
        gtba = _gt(kb, ib, ka, ia)
        sh = (G, j, S, 128)
        if k is None:
            dir_a = None
        else:
            n_a = (lax.broadcasted_iota(jnp.int32, sh, 2) * T
                   + lax.broadcasted_iota(jnp.int32, sh, 0) * (2 * j)
                   + lax.broadcasted_iota(jnp.int32, sh, 1))
            dir_a = ((n_a & k) != 0) if invert else ((n_a & k) == 0)
        if lane_level is not None:
            flip = (lax.broadcasted_iota(jnp.int32, sh, 3)
                    & (1 << lane_level)) != 0
            dir_a = flip != dir_a if dir_a is not None else ~flip
        swap = gtba if dir_a is None else (gtba == dir_a)
        nak = jnp.where(swap, kb, ka)
        nbk = jnp.where(swap, ka, kb)
        nai = jnp.where(swap, ib, ia)
        nbi = jnp.where(swap, ia, ib)
        karr = jnp.concatenate([nak[:, None], nbk[:, None]], axis=1).reshape(T, S, 128)
        iarr = jnp.concatenate([nai[:, None], nbi[:, None]], axis=1).reshape(T, S, 128)
    else:
        js = j // T
        sh = (T, S, 128)
        si = lax.broadcasted_iota(jnp.int32, sh, 1)
        low = (si & js) == 0
        pk = jnp.where(low, jnp.roll(karr, -js, axis=1), jnp.roll(karr, js, axis=1))
        pi = jnp.where(low, jnp.roll(iarr, -js, axis=1), jnp.roll(iarr, js, axis=1))
        gtp = _gt(pk, pi, karr, iarr)
        if k is None:
            d = None
        else:
            n = si * T + lax.broadcasted_iota(jnp.int32, sh, 0)
            d = ((n & k) != 0) if invert else ((n & k) == 0)
        if lane_level is not None:
            flip = (lax.broadcasted_iota(jnp.int32, sh, 2)
                    & (1 << lane_level)) != 0
            d = flip != d if d is not None else ~flip
        winner_here = low if d is None else (low == d)
        take = gtp == winner_here
        karr = jnp.where(take, pk, karr)
        iarr = jnp.where(take, pi, iarr)
    return karr, iarr


def _topk_body(in_ref, out_s_ref, out_i_ref, rk_ref, ri_ref, *,
               n_valid, n_chunks, compute_scores):
    c = pl.program_id(1)

    x = in_ref[0].reshape(_T, 8, 128)
    lane = lax.broadcasted_iota(jnp.int32, (_T, 8, 128), 2)
    if compute_scores:
        p = jnp.where(lane < _L, x, -1.0)
        amax = jnp.max(p, axis=2, keepdims=True)
        argl = jnp.min(jnp.where(p == amax, lane, 128), axis=2, keepdims=True)
        s = jnp.where(argl == 0, 0.0, p)
    else:
        s = x
    key = _f2key(s)
    t_i = lax.broadcasted_iota(jnp.int32, (_T, 8, 128), 0)
    s_i = lax.broadcasted_iota(jnp.int32, (_T, 8, 128), 1)
    rowid = c * _CHUNK + t_i * 8 + s_i
    key = jnp.where(rowid < n_valid, key, _INT_MIN)

    for k in [2, 4, 8, 16, 32, 64, 128, 256, 512]:
        j = k // 2
        while j >= 1:
            key, rowid = _stage(key, rowid, _T, 8, j, k, invert=True)
            j //= 2

    akr, air = key[:, 4:8], rowid[:, 4:8]

    @pl.when(c == 0)
    def _init():
        rk_ref[...] = jnp.full((_T, 4, 128), _INT_MIN, jnp.int32)
        ri_ref[...] = jnp.full((_T, 4, 128), _IDX_PAD, jnp.int32)

    rk, ri = rk_ref[...], ri_ref[...]
    gta = _gt(akr, air, rk, ri)
    wk = jnp.where(gta, akr, rk)
    wi = jnp.where(gta, air, ri)
    for j in [128, 64, 32, 16, 8, 4, 2, 1]:
        wk, wi = _stage(wk, wi, _T, 4, j, None)
    rk_ref[...] = wk
    ri_ref[...] = wi

    @pl.when(c == n_chunks - 1)
    def _emit():
        sc = _key2f(wk)
        for si in range(4):
            out_s_ref[0, si] = sc[:, si, :]
            out_i_ref[0, si] = wi[:, si, :]


def _topk256(x, n_valid, compute_scores):
    B, n, _ = x.shape
    n_chunks = (n + _CHUNK - 1) // _CHUNK
    out = pl.pallas_call(
        functools.partial(_topk_body, n_valid=n_valid, n_chunks=n_chunks,
                          compute_scores=compute_scores),
        grid=(B, n_chunks),
        in_specs=[pl.BlockSpec((1, _CHUNK, 128), lambda b, c: (b, c, 0))],
        out_specs=[pl.BlockSpec((1, 4, _T, 128), lambda b, c: (b, 0, 0, 0)),
                   pl.BlockSpec((1, 4, _T, 128), lambda b, c: (b, 0, 0, 0))],
        out_shape=[jax.ShapeDtypeStruct((B, 4, _T, 128), jnp.float32),
                   jax.ShapeDtypeStruct((B, 4, _T, 128), jnp.int32)],
        scratch_shapes=[pltpu.VMEM((_T, 4, 128), jnp.int32),
                        pltpu.VMEM((_T, 4, 128), jnp.int32)],
    )(x)
    return out[0].reshape(B, 256, 128), out[1].reshape(B, 256, 128)


def _gmerge_body(in_ref, out_s_ref, out_i_ref):
    x = in_ref[0]
    keys = jnp.concatenate(
        [_f2key(x), jnp.full((64, 128), _INT_MIN, jnp.int32)], axis=0)
    keys = keys.reshape(32, 8, 128)
    sh = (32, 8, 128)
    t_i = lax.broadcasted_iota(jnp.int32, sh, 0)
    s_i = lax.broadcasted_iota(jnp.int32, sh, 1)
    l_i = lax.broadcasted_iota(jnp.int32, sh, 2)
    r = t_i * 8 + s_i
    idx = jnp.where(r < 192, r * 128 + l_i, _IDX_PAD)

    for k in [2, 4, 8, 16, 32, 64, 128, 256]:
        j = k // 2
        while j >= 1:
            keys, idx = _stage(keys, idx, 32, 8, j, k, lane_level=0)
            j //= 2

    for lev in range(1, 8):
        d = 1 << (lev - 1)
        low = (l_i & d) == 0
        pk = jnp.where(low, jnp.roll(keys, -d, axis=2), jnp.roll(keys, d, axis=2))
        pi = jnp.where(low, jnp.roll(idx, -d, axis=2), jnp.roll(idx, d, axis=2))
        win = _gt(pk, pi, keys, idx)
        keys = jnp.where(win, pk, keys)
        idx = jnp.where(win, pi, idx)
        for j in [128, 64, 32, 16, 8, 4, 2, 1]:
            keys, idx = _stage(keys, idx, 32, 8, j, None, lane_level=lev)

    sc = _key2f(keys)
    for si in range(8):
        out_s_ref[0, si] = sc[:, si, :]
        out_i_ref[0, si] = idx[:, si, :]


def _gmerge(x):
    B = x.shape[0]
    out = pl.pallas_call(
        _gmerge_body,
        grid=(B,),
        in_specs=[pl.BlockSpec((1, 192, 128), lambda b: (b, 0, 0))],
        out_specs=[pl.BlockSpec((1, 8, 32, 128), lambda b: (b, 0, 0, 0)),
                   pl.BlockSpec((1, 8, 32, 128), lambda b: (b, 0, 0, 0))],
        out_shape=[jax.ShapeDtypeStruct((B, 8, 32, 128), jnp.float32),
                   jax.ShapeDtypeStruct((B, 8, 32, 128), jnp.int32)],
    )(x)
    return out[0].reshape(B, 256, 128), out[1].reshape(B, 256, 128)


_LB = 16


def _decode(r, d):
    h = r[2] - r[0]
    w = r[3] - r[1]
    cy = r[0] + 0.5 * h
    cx = r[1] + 0.5 * w
    dy = d[0] * np.float32(0.1)
    dx = d[1] * np.float32(0.1)
    dh = d[2] * np.float32(0.2)
    dw = d[3] * np.float32(0.2)
    ph = jnp.exp(dh) * h
    pw = jnp.exp(dw) * w
    pcy = dy * h + cy
    pcx = dx * w + cx
    y1 = pcy - 0.5 * ph
    x1 = pcx - 0.5 * pw
    y2 = y1 + ph
    x2 = x1 + pw
    return y1, x1, y2, x2


def _nms_body(s_ref, r_ref, d_ref, out_s_ref, out_b_ref, iou_ref):
    y1, x1, y2, x2 = _decode(r_ref[0], d_ref[0])
    area = jnp.maximum(y2 - y1, 0.0) * jnp.maximum(x2 - x1, 0.0)

    iy1 = jnp.maximum(y1[:, :, None], y1[:, None, :])
    ix1 = jnp.maximum(x1[:, :, None], x1[:, None, :])
    iy2 = jnp.minimum(y2[:, :, None], y2[:, None, :])
    ix2 = jnp.minimum(x2[:, :, None], x2[:, None, :])
    inter = jnp.maximum(iy2 - iy1, 0.0) * jnp.maximum(ix2 - ix1, 0.0)
    union = area[:, :, None] + area[:, None, :] - inter
    iou_ref[...] = inter / jnp.maximum(union, 1e-8)

    s = s_ref[0]
    jlane = lax.broadcasted_iota(jnp.int32, (_LB, 256), 1)

    def body(i, keep):
        row = iou_ref[:, pl.ds(i, 1), :].reshape(_LB, 256)
        keep_i = jnp.sum(jnp.where(jlane == i, keep, 0.0),
                         axis=1, keepdims=True)
        sup = (row > _IOU_THR) & (jlane > i) & (keep_i > 0.0)
        return jnp.where(sup, 0.0, keep)

    keep_f = lax.fori_loop(0, _CAND, body,
                           jnp.where(s > _SCORE_THR, 1.0, 0.0))
    keep = keep_f > 0.0
    lt = (lax.broadcasted_iota(jnp.int32, (256, 256), 0)
          <= lax.broadcasted_iota(jnp.int32, (256, 256), 1)).astype(jnp.float32)
    cum = jnp.dot(keep_f, lt, preferred_element_type=jnp.float32)
    keep = keep & (cum - 1.0 < np.float32(_MAX_TOTAL))
    out_s_ref[0] = jnp.where(keep, s, -1.0)
    out_b_ref[0, 0] = y1
    out_b_ref[0, 1] = x1
    out_b_ref[0, 2] = y2
    out_b_ref[0, 3] = x2


def _nms(s_in, rois_c, deltas_c):
    B = s_in.shape[0]
    return pl.pallas_call(
        _nms_body,
        grid=(B, _LP // _LB),
        in_specs=[
            pl.BlockSpec((1, _LB, 256), lambda b, l: (b, l, 0)),
            pl.BlockSpec((1, 4, _LB, 256), lambda b, l: (b, 0, l, 0)),
            pl.BlockSpec((1, 4, _LB, 256), lambda b, l: (b, 0, l, 0)),
        ],
        out_specs=[pl.BlockSpec((1, _LB, 256), lambda b, l: (b, l, 0)),
                   pl.BlockSpec((1, 4, _LB, 256), lambda b, l: (b, 0, l, 0))],
        out_shape=[jax.ShapeDtypeStruct((B, _LP, 256), jnp.float32),
                   jax.ShapeDtypeStruct((B, 4, _LP, 256), jnp.float32)],
        scratch_shapes=[pltpu.VMEM((_LB, 256, 256), jnp.float32)],
    )(s_in, rois_c, deltas_c)


def kernel(roi_bboxes, pred_deltas, pred_label_probs):
    B = roi_bboxes.shape[0]

    s1, i1 = _topk256(pred_label_probs, _N, compute_scores=True)
    s1 = jnp.transpose(s1[:, :, :_L], (0, 2, 1))
    i1 = jnp.transpose(i1[:, :, :_L], (0, 2, 1))

    s_in = jnp.concatenate(
        [s1, jnp.full((B, _LP - _L, 256), -2.0, jnp.float32)], axis=1)
    idx = jnp.concatenate(
        [i1, jnp.zeros((B, _LP - _L, 256), jnp.int32)], axis=1)
    idx = jnp.clip(idx, 0, _N - 1)

    flat_idx = idx.reshape(B, -1)
    rois_g = jax.vmap(lambda r, i: r[i])(roi_bboxes, flat_idx)
    d_rows = jnp.take_along_axis(pred_deltas, flat_idx[..., None], axis=1)
    d_rows = d_rows.reshape(B, _LP, 256, _L * 4)
    lcls = jnp.minimum(jnp.arange(_LP, dtype=jnp.int32), _L - 1)
    cols = (4 * lcls[None, :, None, None]
            + jnp.arange(4, dtype=jnp.int32)[None, None, None, :])
    deltas_g = jnp.take_along_axis(
        d_rows, jnp.broadcast_to(cols, (B, _LP, 256, 4)), axis=3)

    rois_c = jnp.transpose(rois_g.reshape(B, _LP, 256, 4), (0, 3, 1, 2))
    deltas_c = jnp.transpose(deltas_g, (0, 3, 1, 2))

    out_s, out_b = _nms(s_in, rois_c, deltas_c)

    s2, i2 = _gmerge(out_s.reshape(B, 192, 128))
    top_s = s2[:, :_MAX_TOTAL, 0]
    top_i = i2[:, :_MAX_TOTAL, 0]

    flat_b = jnp.transpose(out_b, (0, 2, 3, 1)).reshape(B, _LP * 256, 4)
    final_bboxes = jnp.take_along_axis(flat_b, top_i[..., None], axis=1)
    labels = (top_i // _CAND).astype(jnp.float32)
    valid = top_s > 0.0
    final_scores = jnp.where(valid, top_s, 0.0)
    final_bboxes = jnp.where(valid[..., None], final_bboxes, 0.0)
    final_labels = jnp.where(valid, labels, 0.0)
    return final_bboxes, final_labels, final_scores


# speedup vs baseline: 6.5819x; 1.1550x over previous
import functools

import numpy as np
import jax
import jax.numpy as jnp
from jax import lax
from jax.experimental import pallas as pl
from jax.experimental.pallas import tpu as pltpu

_L = 91
_LP = 96
_N = 20000
_CAND = 256
_MAX_TOTAL = 200
_SCORE_THR = 0.5
_IOU_THR = 0.5
_CHUNK = 512
_T = 64
_INT_MIN = np.int32(-2**31)
_IDX_PAD = np.int32(2**31 - 1)


def _f2key(s):
    b = lax.bitcast_convert_type(s, jnp.int32)
    return b ^ (lax.shift_right_arithmetic(b, 31) & np.int32(0x7FFFFFFF))


def _key2f(k):
    return lax.bitcast_convert_type(
        k ^ (lax.shift_right_arithmetic(k, 31) & np.int32(0x7FFFFFFF)), jnp.float32)


def _gt(ka, ia, kb, ib):
    return (ka > kb) | ((ka == kb) & (ia < ib))


def _stage(karr, iarr, T, S, j, k, invert=False, lane_level=None):
    if j < T:
        G = T // (2 * j)
        k5 = karr.reshape(G, 2, j, S, 128)
        i5 = iarr.reshape(G, 2, j, S, 128)
        ka, kb = k5[:, 0], k5[:, 1]
        ia, ib = i5[:, 0], i5[:, 1]
        gtba = _gt(kb, ib, ka, ia)
        sh = (G, j, S, 128)
        if k is None:
            dir_a = None
        else:
            n_a = (lax.broadcasted_iota(jnp.int32, sh, 2) * T
                   + lax.broadcasted_iota(jnp.int32, sh, 0) * (2 * j)
                   + lax.broadcasted_iota(jnp.int32, sh, 1))
            dir_a = ((n_a & k) != 0) if invert else ((n_a & k) == 0)
        if lane_level is not None:
            flip = (lax.broadcasted_iota(jnp.int32, sh, 3)
                    & (1 << lane_level)) != 0
            dir_a = flip != dir_a if dir_a is not None else ~flip
        swap = gtba if dir_a is None else (gtba == dir_a)
        nak = jnp.where(swap, kb, ka)
        nbk = jnp.where(swap, ka, kb)
        nai = jnp.where(swap, ib, ia)
        nbi = jnp.where(swap, ia, ib)
        karr = jnp.concatenate([nak[:, None], nbk[:, None]], axis=1).reshape(T, S, 128)
        iarr = jnp.concatenate([nai[:, None], nbi[:, None]], axis=1).reshape(T, S, 128)
    else:
        js = j // T
        sh = (T, S, 128)
        si = lax.broadcasted_iota(jnp.int32, sh, 1)
        low = (si & js) == 0
        pk = jnp.where(low, jnp.roll(karr, -js, axis=1), jnp.roll(karr, js, axis=1))
        pi = jnp.where(low, jnp.roll(iarr, -js, axis=1), jnp.roll(iarr, js, axis=1))
        gtp = _gt(pk, pi, karr, iarr)
        if k is None:
            d = None
        else:
            n = si * T + lax.broadcasted_iota(jnp.int32, sh, 0)
            d = ((n & k) != 0) if invert else ((n & k) == 0)
        if lane_level is not None:
            flip = (lax.broadcasted_iota(jnp.int32, sh, 2)
                    & (1 << lane_level)) != 0
            d = flip != d if d is not None else ~flip
        winner_here = low if d is None else (low == d)
        take = gtp == winner_here
        karr = jnp.where(take, pk, karr)
        iarr = jnp.where(take, pi, iarr)
    return karr, iarr


def _topk_body(in_ref, out_s_ref, out_i_ref, rk_ref, ri_ref, *,
               n_valid, n_chunks, compute_scores):
    c = pl.program_id(1)

    x = in_ref[0].reshape(_T, 8, 128)
    lane = lax.broadcasted_iota(jnp.int32, (_T, 8, 128), 2)
    if compute_scores:
        p = jnp.where(lane < _L, x, -1.0)
        amax = jnp.max(p, axis=2, keepdims=True)
        argl = jnp.min(jnp.where(p == amax, lane, 128), axis=2, keepdims=True)
        s = jnp.where(argl == 0, 0.0, p)
    else:
        s = x
    key = _f2key(s)
    t_i = lax.broadcasted_iota(jnp.int32, (_T, 8, 128), 0)
    s_i = lax.broadcasted_iota(jnp.int32, (_T, 8, 128), 1)
    rowid = c * _CHUNK + t_i * 8 + s_i
    key = jnp.where(rowid < n_valid, key, _INT_MIN)

    for k in [2, 4, 8, 16, 32, 64, 128, 256, 512]:
        j = k // 2
        while j >= 1:
            key, rowid = _stage(key, rowid, _T, 8, j, k, invert=True)
            j //= 2

    akr, air = key[:, 4:8], rowid[:, 4:8]

    @pl.when(c == 0)
    def _init():
        rk_ref[...] = jnp.full((_T, 4, 128), _INT_MIN, jnp.int32)
        ri_ref[...] = jnp.full((_T, 4, 128), _IDX_PAD, jnp.int32)

    rk, ri = rk_ref[...], ri_ref[...]
    gta = _gt(akr, air, rk, ri)
    wk = jnp.where(gta, akr, rk)
    wi = jnp.where(gta, air, ri)
    for j in [128, 64, 32, 16, 8, 4, 2, 1]:
        wk, wi = _stage(wk, wi, _T, 4, j, None)
    rk_ref[...] = wk
    ri_ref[...] = wi

    @pl.when(c == n_chunks - 1)
    def _emit():
        sc = _key2f(wk)
        for si in range(4):
            out_s_ref[0, si] = sc[:, si, :]
            out_i_ref[0, si] = wi[:, si, :]


def _topk256(x, n_valid, compute_scores):
    B, n, _ = x.shape
    n_chunks = (n + _CHUNK - 1) // _CHUNK
    out = pl.pallas_call(
        functools.partial(_topk_body, n_valid=n_valid, n_chunks=n_chunks,
                          compute_scores=compute_scores),
        grid=(B, n_chunks),
        in_specs=[pl.BlockSpec((1, _CHUNK, 128), lambda b, c: (b, c, 0))],
        out_specs=[pl.BlockSpec((1, 4, _T, 128), lambda b, c: (b, 0, 0, 0)),
                   pl.BlockSpec((1, 4, _T, 128), lambda b, c: (b, 0, 0, 0))],
        out_shape=[jax.ShapeDtypeStruct((B, 4, _T, 128), jnp.float32),
                   jax.ShapeDtypeStruct((B, 4, _T, 128), jnp.int32)],
        scratch_shapes=[pltpu.VMEM((_T, 4, 128), jnp.int32),
                        pltpu.VMEM((_T, 4, 128), jnp.int32)],
    )(x)
    return out[0].reshape(B, 256, 128), out[1].reshape(B, 256, 128)


def _gmerge_body(in_ref, out_s_ref, out_i_ref):
    x = in_ref[0]
    keys = jnp.concatenate(
        [_f2key(x), jnp.full((64, 128), _INT_MIN, jnp.int32)], axis=0)
    keys = keys.reshape(32, 8, 128)
    sh = (32, 8, 128)
    t_i = lax.broadcasted_iota(jnp.int32, sh, 0)
    s_i = lax.broadcasted_iota(jnp.int32, sh, 1)
    l_i = lax.broadcasted_iota(jnp.int32, sh, 2)
    r = t_i * 8 + s_i
    idx = jnp.where(r < 192, r * 128 + l_i, _IDX_PAD)

    for k in [2, 4, 8, 16, 32, 64, 128, 256]:
        j = k // 2
        while j >= 1:
            keys, idx = _stage(keys, idx, 32, 8, j, k, lane_level=0)
            j //= 2

    for lev in range(1, 8):
        d = 1 << (lev - 1)
        low = (l_i & d) == 0
        pk = jnp.where(low, jnp.roll(keys, -d, axis=2), jnp.roll(keys, d, axis=2))
        pi = jnp.where(low, jnp.roll(idx, -d, axis=2), jnp.roll(idx, d, axis=2))
        win = _gt(pk, pi, keys, idx)
        keys = jnp.where(win, pk, keys)
        idx = jnp.where(win, pi, idx)
        for j in [128, 64, 32, 16, 8, 4, 2, 1]:
            keys, idx = _stage(keys, idx, 32, 8, j, None, lane_level=lev)

    sc = _key2f(keys)
    for si in range(8):
        out_s_ref[0, si] = sc[:, si, :]
        out_i_ref[0, si] = idx[:, si, :]


def _gmerge(x):
    B = x.shape[0]
    out = pl.pallas_call(
        _gmerge_body,
        grid=(B,),
        in_specs=[pl.BlockSpec((1, 192, 128), lambda b: (b, 0, 0))],
        out_specs=[pl.BlockSpec((1, 8, 32, 128), lambda b: (b, 0, 0, 0)),
                   pl.BlockSpec((1, 8, 32, 128), lambda b: (b, 0, 0, 0))],
        out_shape=[jax.ShapeDtypeStruct((B, 8, 32, 128), jnp.float32),
                   jax.ShapeDtypeStruct((B, 8, 32, 128), jnp.int32)],
    )(x)
    return out[0].reshape(B, 256, 128), out[1].reshape(B, 256, 128)


_LB = 48


def _decode(r, d):
    h = r[2] - r[0]
    w = r[3] - r[1]
    cy = r[0] + 0.5 * h
    cx = r[1] + 0.5 * w
    dy = d[0] * np.float32(0.1)
    dx = d[1] * np.float32(0.1)
    dh = d[2] * np.float32(0.2)
    dw = d[3] * np.float32(0.2)
    ph = jnp.exp(dh) * h
    pw = jnp.exp(dw) * w
    pcy = dy * h + cy
    pcx = dx * w + cx
    y1 = pcy - 0.5 * ph
    x1 = pcx - 0.5 * pw
    y2 = y1 + ph
    x2 = x1 + pw
    return y1, x1, y2, x2


def _nms_body(s_ref, r_ref, d_ref, out_s_ref, out_b_ref, iou_ref):
    y1, x1, y2, x2 = _decode(r_ref[0], d_ref[0])
    area = jnp.maximum(y2 - y1, 0.0) * jnp.maximum(x2 - x1, 0.0)

    iy1 = jnp.maximum(y1[:, :, None], y1[:, None, :])
    ix1 = jnp.maximum(x1[:, :, None], x1[:, None, :])
    iy2 = jnp.minimum(y2[:, :, None], y2[:, None, :])
    ix2 = jnp.minimum(x2[:, :, None], x2[:, None, :])
    inter = jnp.maximum(iy2 - iy1, 0.0) * jnp.maximum(ix2 - ix1, 0.0)
    union = area[:, :, None] + area[:, None, :] - inter
    iou_ref[...] = inter / jnp.maximum(union, 1e-8)

    s = s_ref[0]
    jlane = lax.broadcasted_iota(jnp.int32, (_LB, 256), 1)

    def body(i, keep):
        row = iou_ref[:, pl.ds(i, 1), :].reshape(_LB, 256)
        keep_i = jnp.sum(jnp.where(jlane == i, keep, 0.0),
                         axis=1, keepdims=True)
        sup = (row > _IOU_THR) & (jlane > i) & (keep_i > 0.0)
        return jnp.where(sup, 0.0, keep)

    keep_f = lax.fori_loop(0, _CAND, body,
                           jnp.where(s > _SCORE_THR, 1.0, 0.0))
    keep = keep_f > 0.0
    lt = (lax.broadcasted_iota(jnp.int32, (256, 256), 0)
          <= lax.broadcasted_iota(jnp.int32, (256, 256), 1)).astype(jnp.float32)
    cum = jnp.dot(keep_f, lt, preferred_element_type=jnp.float32)
    keep = keep & (cum - 1.0 < np.float32(_MAX_TOTAL))
    out_s_ref[0] = jnp.where(keep, s, -1.0)
    out_b_ref[0, 0] = y1
    out_b_ref[0, 1] = x1
    out_b_ref[0, 2] = y2
    out_b_ref[0, 3] = x2


def _nms(s_in, rois_c, deltas_c):
    B = s_in.shape[0]
    return pl.pallas_call(
        _nms_body,
        grid=(B, _LP // _LB),
        in_specs=[
            pl.BlockSpec((1, _LB, 256), lambda b, l: (b, l, 0)),
            pl.BlockSpec((1, 4, _LB, 256), lambda b, l: (b, 0, l, 0)),
            pl.BlockSpec((1, 4, _LB, 256), lambda b, l: (b, 0, l, 0)),
        ],
        out_specs=[pl.BlockSpec((1, _LB, 256), lambda b, l: (b, l, 0)),
                   pl.BlockSpec((1, 4, _LB, 256), lambda b, l: (b, 0, l, 0))],
        out_shape=[jax.ShapeDtypeStruct((B, _LP, 256), jnp.float32),
                   jax.ShapeDtypeStruct((B, 4, _LP, 256), jnp.float32)],
        scratch_shapes=[pltpu.VMEM((_LB, 256, 256), jnp.float32)],
    )(s_in, rois_c, deltas_c)


def kernel(roi_bboxes, pred_deltas, pred_label_probs):
    B = roi_bboxes.shape[0]

    s1, i1 = _topk256(pred_label_probs, _N, compute_scores=True)
    s1 = jnp.transpose(s1[:, :, :_L], (0, 2, 1))
    i1 = jnp.transpose(i1[:, :, :_L], (0, 2, 1))

    s_in = jnp.concatenate(
        [s1, jnp.full((B, _LP - _L, 256), -2.0, jnp.float32)], axis=1)
    idx = jnp.concatenate(
        [i1, jnp.zeros((B, _LP - _L, 256), jnp.int32)], axis=1)
    idx = jnp.clip(idx, 0, _N - 1)

    flat_idx = idx.reshape(B, -1)
    rois_g = jax.vmap(lambda r, i: r[i])(roi_bboxes, flat_idx)
    d_rows = jnp.take_along_axis(pred_deltas, flat_idx[..., None], axis=1)
    d_rows = d_rows.reshape(B, _LP, 256, _L * 4)
    lcls = jnp.minimum(jnp.arange(_LP, dtype=jnp.int32), _L - 1)
    cols = (4 * lcls[None, :, None, None]
            + jnp.arange(4, dtype=jnp.int32)[None, None, None, :])
    deltas_g = jnp.take_along_axis(
        d_rows, jnp.broadcast_to(cols, (B, _LP, 256, 4)), axis=3)

    rois_c = jnp.transpose(rois_g.reshape(B, _LP, 256, 4), (0, 3, 1, 2))
    deltas_c = jnp.transpose(deltas_g, (0, 3, 1, 2))

    out_s, out_b = _nms(s_in, rois_c, deltas_c)

    s2, i2 = _gmerge(out_s.reshape(B, 192, 128))
    top_s = s2[:, :_MAX_TOTAL, 0]
    top_i = i2[:, :_MAX_TOTAL, 0]

    flat_b = jnp.transpose(out_b, (0, 2, 3, 1)).reshape(B, _LP * 256, 4)
    final_bboxes = jnp.take_along_axis(flat_b, top_i[..., None], axis=1)
    labels = (top_i // _CAND).astype(jnp.float32)
    valid = top_s > 0.0
    final_scores = jnp.where(valid, top_s, 0.0)
    final_bboxes = jnp.where(valid[..., None], final_bboxes, 0.0)
    final_labels = jnp.where(valid, labels, 0.0)
    return final_bboxes, final_labels, final_scores


# speedup vs baseline: 7.6988x; 1.1697x over previous
import functools

import numpy as np
import jax
import jax.numpy as jnp
from jax import lax
from jax.experimental import pallas as pl
from jax.experimental.pallas import tpu as pltpu

_L = 91
_LP = 96
_N = 20000
_CAND = 256
_MAX_TOTAL = 200
_SCORE_THR = 0.5
_IOU_THR = 0.5
_CHUNK = 512
_T = 64
_INT_MIN = np.int32(-2**31)
_IDX_PAD = np.int32(2**31 - 1)


def _f2key(s):
    b = lax.bitcast_convert_type(s, jnp.int32)
    return b ^ (lax.shift_right_arithmetic(b, 31) & np.int32(0x7FFFFFFF))


def _key2f(k):
    return lax.bitcast_convert_type(
        k ^ (lax.shift_right_arithmetic(k, 31) & np.int32(0x7FFFFFFF)), jnp.float32)


def _gt(ka, ia, kb, ib):
    return (ka > kb) | ((ka == kb) & (ia < ib))


def _stage(karr, iarr, T, S, j, k, invert=False, lane_level=None):
    if j < T:
        G = T // (2 * j)
        k5 = karr.reshape(G, 2, j, S, 128)
        i5 = iarr.reshape(G, 2, j, S, 128)
        ka, kb = k5[:, 0], k5[:, 1]
        ia, ib = i5[:, 0], i5[:, 1]
        gtba = _gt(kb, ib, ka, ia)
        sh = (G, j, S, 128)
        if k is None:
            dir_a = None
        else:
            n_a = (lax.broadcasted_iota(jnp.int32, sh, 2) * T
                   + lax.broadcasted_iota(jnp.int32, sh, 0) * (2 * j)
                   + lax.broadcasted_iota(jnp.int32, sh, 1))
            dir_a = ((n_a & k) != 0) if invert else ((n_a & k) == 0)
        if lane_level is not None:
            flip = (lax.broadcasted_iota(jnp.int32, sh, 3)
                    & (1 << lane_level)) != 0
            dir_a = flip != dir_a if dir_a is not None else ~flip
        swap = gtba if dir_a is None else (gtba == dir_a)
        nak = jnp.where(swap, kb, ka)
        nbk = jnp.where(swap, ka, kb)
        nai = jnp.where(swap, ib, ia)
        nbi = jnp.where(swap, ia, ib)
        karr = jnp.concatenate([nak[:, None], nbk[:, None]], axis=1).reshape(T, S, 128)
        iarr = jnp.concatenate([nai[:, None], nbi[:, None]], axis=1).reshape(T, S, 128)
    else:
        js = j // T
        sh = (T, S, 128)
        si = lax.broadcasted_iota(jnp.int32, sh, 1)
        low = (si & js) == 0
        pk = jnp.where(low, jnp.roll(karr, -js, axis=1), jnp.roll(karr, js, axis=1))
        pi = jnp.where(low, jnp.roll(iarr, -js, axis=1), jnp.roll(iarr, js, axis=1))
        gtp = _gt(pk, pi, karr, iarr)
        if k is None:
            d = None
        else:
            n = si * T + lax.broadcasted_iota(jnp.int32, sh, 0)
            d = ((n & k) != 0) if invert else ((n & k) == 0)
        if lane_level is not None:
            flip = (lax.broadcasted_iota(jnp.int32, sh, 2)
                    & (1 << lane_level)) != 0
            d = flip != d if d is not None else ~flip
        winner_here = low if d is None else (low == d)
        take = gtp == winner_here
        karr = jnp.where(take, pk, karr)
        iarr = jnp.where(take, pi, iarr)
    return karr, iarr


def _topk_body(in_ref, out_s_ref, out_i_ref, rk_ref, ri_ref, *,
               n_valid, n_chunks, compute_scores):
    c = pl.program_id(1)

    x = in_ref[0].reshape(_T, 8, 128)
    lane = lax.broadcasted_iota(jnp.int32, (_T, 8, 128), 2)
    if compute_scores:
        p = jnp.where(lane < _L, x, -1.0)
        amax = jnp.max(p, axis=2, keepdims=True)
        argl = jnp.min(jnp.where(p == amax, lane, 128), axis=2, keepdims=True)
        s = jnp.where(argl == 0, 0.0, p)
    else:
        s = x
    key = _f2key(s)
    t_i = lax.broadcasted_iota(jnp.int32, (_T, 8, 128), 0)
    s_i = lax.broadcasted_iota(jnp.int32, (_T, 8, 128), 1)
    rowid = c * _CHUNK + t_i * 8 + s_i
    key = jnp.where(rowid < n_valid, key, _INT_MIN)

    for k in [2, 4, 8, 16, 32, 64, 128, 256, 512]:
        j = k // 2
        while j >= 1:
            key, rowid = _stage(key, rowid, _T, 8, j, k, invert=True)
            j //= 2

    akr, air = key[:, 4:8], rowid[:, 4:8]

    @pl.when(c == 0)
    def _init():
        rk_ref[...] = jnp.full((_T, 4, 128), _INT_MIN, jnp.int32)
        ri_ref[...] = jnp.full((_T, 4, 128), _IDX_PAD, jnp.int32)

    rk, ri = rk_ref[...], ri_ref[...]
    gta = _gt(akr, air, rk, ri)
    wk = jnp.where(gta, akr, rk)
    wi = jnp.where(gta, air, ri)
    for j in [128, 64, 32, 16, 8, 4, 2, 1]:
        wk, wi = _stage(wk, wi, _T, 4, j, None)
    rk_ref[...] = wk
    ri_ref[...] = wi

    @pl.when(c == n_chunks - 1)
    def _emit():
        sc = _key2f(wk)
        for si in range(4):
            out_s_ref[0, si] = sc[:, si, :]
            out_i_ref[0, si] = wi[:, si, :]


def _topk256(x, n_valid, compute_scores):
    B, n, _ = x.shape
    n_chunks = (n + _CHUNK - 1) // _CHUNK
    out = pl.pallas_call(
        functools.partial(_topk_body, n_valid=n_valid, n_chunks=n_chunks,
                          compute_scores=compute_scores),
        grid=(B, n_chunks),
        in_specs=[pl.BlockSpec((1, _CHUNK, 128), lambda b, c: (b, c, 0))],
        out_specs=[pl.BlockSpec((1, 4, _T, 128), lambda b, c: (b, 0, 0, 0)),
                   pl.BlockSpec((1, 4, _T, 128), lambda b, c: (b, 0, 0, 0))],
        out_shape=[jax.ShapeDtypeStruct((B, 4, _T, 128), jnp.float32),
                   jax.ShapeDtypeStruct((B, 4, _T, 128), jnp.int32)],
        scratch_shapes=[pltpu.VMEM((_T, 4, 128), jnp.int32),
                        pltpu.VMEM((_T, 4, 128), jnp.int32)],
    )(x)
    return out[0].reshape(B, 256, 128), out[1].reshape(B, 256, 128)


def _gmerge_body(in_ref, out_s_ref, out_i_ref):
    x = in_ref[0]
    keys = jnp.concatenate(
        [_f2key(x), jnp.full((64, 128), _INT_MIN, jnp.int32)], axis=0)
    keys = keys.reshape(32, 8, 128)
    sh = (32, 8, 128)
    t_i = lax.broadcasted_iota(jnp.int32, sh, 0)
    s_i = lax.broadcasted_iota(jnp.int32, sh, 1)
    l_i = lax.broadcasted_iota(jnp.int32, sh, 2)
    r = t_i * 8 + s_i
    idx = jnp.where(r < 192, r * 128 + l_i, _IDX_PAD)

    for k in [2, 4, 8, 16, 32, 64, 128, 256]:
        j = k // 2
        while j >= 1:
            keys, idx = _stage(keys, idx, 32, 8, j, k, lane_level=0)
            j //= 2

    for lev in range(1, 8):
        d = 1 << (lev - 1)
        low = (l_i & d) == 0
        pk = jnp.where(low, jnp.roll(keys, -d, axis=2), jnp.roll(keys, d, axis=2))
        pi = jnp.where(low, jnp.roll(idx, -d, axis=2), jnp.roll(idx, d, axis=2))
        win = _gt(pk, pi, keys, idx)
        keys = jnp.where(win, pk, keys)
        idx = jnp.where(win, pi, idx)
        for j in [128, 64, 32, 16, 8, 4, 2, 1]:
            keys, idx = _stage(keys, idx, 32, 8, j, None, lane_level=lev)

    sc = _key2f(keys)
    for si in range(8):
        out_s_ref[0, si] = sc[:, si, :]
        out_i_ref[0, si] = idx[:, si, :]


def _gmerge(x):
    B = x.shape[0]
    out = pl.pallas_call(
        _gmerge_body,
        grid=(B,),
        in_specs=[pl.BlockSpec((1, 192, 128), lambda b: (b, 0, 0))],
        out_specs=[pl.BlockSpec((1, 8, 32, 128), lambda b: (b, 0, 0, 0)),
                   pl.BlockSpec((1, 8, 32, 128), lambda b: (b, 0, 0, 0))],
        out_shape=[jax.ShapeDtypeStruct((B, 8, 32, 128), jnp.float32),
                   jax.ShapeDtypeStruct((B, 8, 32, 128), jnp.int32)],
    )(x)
    return out[0].reshape(B, 256, 128), out[1].reshape(B, 256, 128)


_LB = 48


def _decode(r, d):
    h = r[2] - r[0]
    w = r[3] - r[1]
    cy = r[0] + 0.5 * h
    cx = r[1] + 0.5 * w
    dy = d[0] * np.float32(0.1)
    dx = d[1] * np.float32(0.1)
    dh = d[2] * np.float32(0.2)
    dw = d[3] * np.float32(0.2)
    ph = jnp.exp(dh) * h
    pw = jnp.exp(dw) * w
    pcy = dy * h + cy
    pcx = dx * w + cx
    y1 = pcy - 0.5 * ph
    x1 = pcx - 0.5 * pw
    y2 = y1 + ph
    x2 = x1 + pw
    return y1, x1, y2, x2


def _nms_body(s_ref, r_ref, d_ref, out_s_ref, out_b_ref, iou_ref):
    y1, x1, y2, x2 = _decode(r_ref[0], d_ref[0])
    area = jnp.maximum(y2 - y1, 0.0) * jnp.maximum(x2 - x1, 0.0)

    iy1 = jnp.maximum(y1[:, :, None], y1[:, None, :])
    ix1 = jnp.maximum(x1[:, :, None], x1[:, None, :])
    iy2 = jnp.minimum(y2[:, :, None], y2[:, None, :])
    ix2 = jnp.minimum(x2[:, :, None], x2[:, None, :])
    inter = jnp.maximum(iy2 - iy1, 0.0) * jnp.maximum(ix2 - ix1, 0.0)
    union = area[:, :, None] + area[:, None, :] - inter
    iou_ref[...] = inter / jnp.maximum(union, 1e-8)

    s = s_ref[0]
    jlane = lax.broadcasted_iota(jnp.int32, (_LB, 256), 1)

    def body(i, keep):
        row = iou_ref[:, pl.ds(i, 1), :].reshape(_LB, 256)
        keep_i = jnp.sum(jnp.where(jlane == i, keep, 0.0),
                         axis=1, keepdims=True)
        sup = (row > _IOU_THR) & (jlane > i) & (keep_i > 0.0)
        return jnp.where(sup, 0.0, keep)

    keep_f = lax.fori_loop(0, _CAND, body,
                           jnp.where(s > _SCORE_THR, 1.0, 0.0))
    keep = keep_f > 0.0
    lt = (lax.broadcasted_iota(jnp.int32, (256, 256), 0)
          <= lax.broadcasted_iota(jnp.int32, (256, 256), 1)).astype(jnp.float32)
    cum = jnp.dot(keep_f, lt, preferred_element_type=jnp.float32)
    keep = keep & (cum - 1.0 < np.float32(_MAX_TOTAL))
    out_s_ref[0] = jnp.where(keep, s, -1.0)
    out_b_ref[0, 0] = y1
    out_b_ref[0, 1] = x1
    out_b_ref[0, 2] = y2
    out_b_ref[0, 3] = x2


def _nms(s_in, rois_c, deltas_c):
    B = s_in.shape[0]
    return pl.pallas_call(
        _nms_body,
        grid=(B, _LP // _LB),
        in_specs=[
            pl.BlockSpec((1, _LB, 256), lambda b, l: (b, l, 0)),
            pl.BlockSpec((1, 4, _LB, 256), lambda b, l: (b, 0, l, 0)),
            pl.BlockSpec((1, 4, _LB, 256), lambda b, l: (b, 0, l, 0)),
        ],
        out_specs=[pl.BlockSpec((1, _LB, 256), lambda b, l: (b, l, 0)),
                   pl.BlockSpec((1, 4, _LB, 256), lambda b, l: (b, 0, l, 0))],
        out_shape=[jax.ShapeDtypeStruct((B, _LP, 256), jnp.float32),
                   jax.ShapeDtypeStruct((B, 4, _LP, 256), jnp.float32)],
        scratch_shapes=[pltpu.VMEM((_LB, 256, 256), jnp.float32)],
    )(s_in, rois_c, deltas_c)


def kernel(roi_bboxes, pred_deltas, pred_label_probs):
    B = roi_bboxes.shape[0]

    s1, i1 = _topk256(pred_label_probs, _N, compute_scores=True)
    s1 = jnp.transpose(s1[:, :, :_L], (0, 2, 1))
    i1 = jnp.transpose(i1[:, :, :_L], (0, 2, 1))

    s_in = jnp.concatenate(
        [s1, jnp.full((B, _LP - _L, 256), -2.0, jnp.float32)], axis=1)
    idx = jnp.concatenate(
        [i1, jnp.zeros((B, _LP - _L, 256), jnp.int32)], axis=1)
    idx = jnp.clip(idx, 0, _N - 1)

    flat_idx = idx.reshape(B, -1)
    rois_g = jax.vmap(lambda r, i: r[i])(roi_bboxes, flat_idx)
    lcls = jnp.minimum(jnp.arange(_LP, dtype=jnp.int32), _L - 1)
    cols = (4 * lcls[None, :, None, None]
            + jnp.arange(4, dtype=jnp.int32)[None, None, None, :])
    cols = jnp.broadcast_to(cols, (B, _LP, 256, 4)).reshape(B, -1)
    rows = jnp.broadcast_to(idx[..., None], (B, _LP, 256, 4)).reshape(B, -1)
    deltas_g = jax.vmap(lambda d, r, c: d[r, c])(pred_deltas, rows, cols)

    rois_c = jnp.transpose(rois_g.reshape(B, _LP, 256, 4), (0, 3, 1, 2))
    deltas_c = jnp.transpose(deltas_g.reshape(B, _LP, 256, 4), (0, 3, 1, 2))

    out_s, out_b = _nms(s_in, rois_c, deltas_c)

    s2, i2 = _gmerge(out_s.reshape(B, 192, 128))
    top_s = s2[:, :_MAX_TOTAL, 0]
    top_i = i2[:, :_MAX_TOTAL, 0]

    flat_b = jnp.transpose(out_b, (0, 2, 3, 1)).reshape(B, _LP * 256, 4)
    final_bboxes = jnp.take_along_axis(flat_b, top_i[..., None], axis=1)
    labels = (top_i // _CAND).astype(jnp.float32)
    valid = top_s > 0.0
    final_scores = jnp.where(valid, top_s, 0.0)
    final_bboxes = jnp.where(valid[..., None], final_bboxes, 0.0)
    final_labels = jnp.where(valid, labels, 0.0)
    return final_bboxes, final_labels, final_scores


# speedup vs baseline: 7.7416x; 1.0056x over previous
import functools

import numpy as np
import jax
import jax.numpy as jnp
from jax import lax
from jax.experimental import pallas as pl
from jax.experimental.pallas import tpu as pltpu

_L = 91
_LP = 96
_N = 20000
_CAND = 256
_MAX_TOTAL = 200
_SCORE_THR = 0.5
_IOU_THR = 0.5
_CHUNK = 512
_T = 64
_INT_MIN = np.int32(-2**31)
_IDX_PAD = np.int32(2**31 - 1)


def _f2key(s):
    b = lax.bitcast_convert_type(s, jnp.int32)
    return b ^ (lax.shift_right_arithmetic(b, 31) & np.int32(0x7FFFFFFF))


def _key2f(k):
    return lax.bitcast_convert_type(
        k ^ (lax.shift_right_arithmetic(k, 31) & np.int32(0x7FFFFFFF)), jnp.float32)


def _gt(ka, ia, kb, ib):
    return (ka > kb) | ((ka == kb) & (ia < ib))


def _stage(karr, iarr, T, S, j, k, invert=False, lane_level=None):
    if j < T:
        G = T // (2 * j)
        k5 = karr.reshape(G, 2, j, S, 128)
        i5 = iarr.reshape(G, 2, j, S, 128)
        ka, kb = k5[:, 0], k5[:, 1]
        ia, ib = i5[:, 0], i5[:, 1]
        gtba = _gt(kb, ib, ka, ia)
        sh = (G, j, S, 128)
        if k is None:
            dir_a = None
        else:
            n_a = (lax.broadcasted_iota(jnp.int32, sh, 2) * T
                   + lax.broadcasted_iota(jnp.int32, sh, 0) * (2 * j)
                   + lax.broadcasted_iota(jnp.int32, sh, 1))
            dir_a = ((n_a & k) != 0) if invert else ((n_a & k) == 0)
        if lane_level is not None:
            flip = (lax.broadcasted_iota(jnp.int32, sh, 3)
                    & (1 << lane_level)) != 0
            dir_a = flip != dir_a if dir_a is not None else ~flip
        swap = gtba if dir_a is None else (gtba == dir_a)
        nak = jnp.where(swap, kb, ka)
        nbk = jnp.where(swap, ka, kb)
        nai = jnp.where(swap, ib, ia)
        nbi = jnp.where(swap, ia, ib)
        karr = jnp.concatenate([nak[:, None], nbk[:, None]], axis=1).reshape(T, S, 128)
        iarr = jnp.concatenate([nai[:, None], nbi[:, None]], axis=1).reshape(T, S, 128)
    else:
        js = j // T
        sh = (T, S, 128)
        si = lax.broadcasted_iota(jnp.int32, sh, 1)
        low = (si & js) == 0
        pk = jnp.where(low, jnp.roll(karr, -js, axis=1), jnp.roll(karr, js, axis=1))
        pi = jnp.where(low, jnp.roll(iarr, -js, axis=1), jnp.roll(iarr, js, axis=1))
        gtp = _gt(pk, pi, karr, iarr)
        if k is None:
            d = None
        else:
            n = si * T + lax.broadcasted_iota(jnp.int32, sh, 0)
            d = ((n & k) != 0) if invert else ((n & k) == 0)
        if lane_level is not None:
            flip = (lax.broadcasted_iota(jnp.int32, sh, 2)
                    & (1 << lane_level)) != 0
            d = flip != d if d is not None else ~flip
        winner_here = low if d is None else (low == d)
        take = gtp == winner_here
        karr = jnp.where(take, pk, karr)
        iarr = jnp.where(take, pi, iarr)
    return karr, iarr


def _topk_body(in_ref, out_s_ref, out_i_ref, rk_ref, ri_ref, *,
               n_valid, n_chunks, compute_scores):
    c = pl.program_id(1)

    x = in_ref[0].reshape(_T, 8, 128)
    lane = lax.broadcasted_iota(jnp.int32, (_T, 8, 128), 2)
    if compute_scores:
        p = jnp.where(lane < _L, x, -1.0)
        amax = jnp.max(p, axis=2, keepdims=True)
        argl = jnp.min(jnp.where(p == amax, lane, 128), axis=2, keepdims=True)
        s = jnp.where(argl == 0, 0.0, p)
    else:
        s = x
    key = _f2key(s)
    t_i = lax.broadcasted_iota(jnp.int32, (_T, 8, 128), 0)
    s_i = lax.broadcasted_iota(jnp.int32, (_T, 8, 128), 1)
    rowid = c * _CHUNK + t_i * 8 + s_i
    key = jnp.where(rowid < n_valid, key, _INT_MIN)

    for k in [2, 4, 8, 16, 32, 64, 128, 256, 512]:
        j = k // 2
        while j >= 1:
            key, rowid = _stage(key, rowid, _T, 8, j, k, invert=True)
            j //= 2

    akr, air = key[:, 4:8], rowid[:, 4:8]

    @pl.when(c == 0)
    def _init():
        rk_ref[...] = jnp.full((_T, 4, 128), _INT_MIN, jnp.int32)
        ri_ref[...] = jnp.full((_T, 4, 128), _IDX_PAD, jnp.int32)

    rk, ri = rk_ref[...], ri_ref[...]
    gta = _gt(akr, air, rk, ri)
    wk = jnp.where(gta, akr, rk)
    wi = jnp.where(gta, air, ri)
    for j in [128, 64, 32, 16, 8, 4, 2, 1]:
        wk, wi = _stage(wk, wi, _T, 4, j, None)
    rk_ref[...] = wk
    ri_ref[...] = wi

    @pl.when(c == n_chunks - 1)
    def _emit():
        sc = _key2f(wk)
        for si in range(4):
            out_s_ref[0, si] = sc[:, si, :]
            out_i_ref[0, si] = wi[:, si, :]


def _topk256(x, n_valid, compute_scores):
    B, n, _ = x.shape
    n_chunks = (n + _CHUNK - 1) // _CHUNK
    out = pl.pallas_call(
        functools.partial(_topk_body, n_valid=n_valid, n_chunks=n_chunks,
                          compute_scores=compute_scores),
        grid=(B, n_chunks),
        in_specs=[pl.BlockSpec((1, _CHUNK, 128), lambda b, c: (b, c, 0))],
        out_specs=[pl.BlockSpec((1, 4, _T, 128), lambda b, c: (b, 0, 0, 0)),
                   pl.BlockSpec((1, 4, _T, 128), lambda b, c: (b, 0, 0, 0))],
        out_shape=[jax.ShapeDtypeStruct((B, 4, _T, 128), jnp.float32),
                   jax.ShapeDtypeStruct((B, 4, _T, 128), jnp.int32)],
        scratch_shapes=[pltpu.VMEM((_T, 4, 128), jnp.int32),
                        pltpu.VMEM((_T, 4, 128), jnp.int32)],
    )(x)
    return out[0].reshape(B, 256, 128), out[1].reshape(B, 256, 128)


def _gmerge_body(in_ref, out_s_ref, out_i_ref):
    x = in_ref[0]
    keys = jnp.concatenate(
        [_f2key(x), jnp.full((64, 128), _INT_MIN, jnp.int32)], axis=0)
    keys = keys.reshape(32, 8, 128)
    sh = (32, 8, 128)
    t_i = lax.broadcasted_iota(jnp.int32, sh, 0)
    s_i = lax.broadcasted_iota(jnp.int32, sh, 1)
    l_i = lax.broadcasted_iota(jnp.int32, sh, 2)
    r = t_i * 8 + s_i
    idx = jnp.where(r < 192, r * 128 + l_i, _IDX_PAD)

    for k in [2, 4, 8, 16, 32, 64, 128, 256]:
        j = k // 2
        while j >= 1:
            keys, idx = _stage(keys, idx, 32, 8, j, k, lane_level=0)
            j //= 2

    for lev in range(1, 8):
        d = 1 << (lev - 1)
        low = (l_i & d) == 0
        pk = jnp.where(low, jnp.roll(keys, -d, axis=2), jnp.roll(keys, d, axis=2))
        pi = jnp.where(low, jnp.roll(idx, -d, axis=2), jnp.roll(idx, d, axis=2))
        win = _gt(pk, pi, keys, idx)
        keys = jnp.where(win, pk, keys)
        idx = jnp.where(win, pi, idx)
        for j in [128, 64, 32, 16, 8, 4, 2, 1]:
            keys, idx = _stage(keys, idx, 32, 8, j, None, lane_level=lev)

    sc = _key2f(keys)
    for si in range(8):
        out_s_ref[0, si] = sc[:, si, :]
        out_i_ref[0, si] = idx[:, si, :]


def _gmerge(x):
    B = x.shape[0]
    out = pl.pallas_call(
        _gmerge_body,
        grid=(B,),
        in_specs=[pl.BlockSpec((1, 192, 128), lambda b: (b, 0, 0))],
        out_specs=[pl.BlockSpec((1, 8, 32, 128), lambda b: (b, 0, 0, 0)),
                   pl.BlockSpec((1, 8, 32, 128), lambda b: (b, 0, 0, 0))],
        out_shape=[jax.ShapeDtypeStruct((B, 8, 32, 128), jnp.float32),
                   jax.ShapeDtypeStruct((B, 8, 32, 128), jnp.int32)],
    )(x)
    return out[0].reshape(B, 256, 128), out[1].reshape(B, 256, 128)


_LB = 48


def _decode(r, d):
    h = r[2] - r[0]
    w = r[3] - r[1]
    cy = r[0] + 0.5 * h
    cx = r[1] + 0.5 * w
    dy = d[0] * np.float32(0.1)
    dx = d[1] * np.float32(0.1)
    dh = d[2] * np.float32(0.2)
    dw = d[3] * np.float32(0.2)
    ph = jnp.exp(dh) * h
    pw = jnp.exp(dw) * w
    pcy = dy * h + cy
    pcx = dx * w + cx
    y1 = pcy - 0.5 * ph
    x1 = pcx - 0.5 * pw
    y2 = y1 + ph
    x2 = x1 + pw
    return y1, x1, y2, x2


def _nms_body(s_ref, r_ref, d_ref, out_s_ref, out_b_ref, iou0_ref, iou1_ref):
    y1, x1, y2, x2 = _decode(r_ref[0], d_ref[0])
    area = jnp.maximum(y2 - y1, 0.0) * jnp.maximum(x2 - x1, 0.0)

    def _iou(ci, cj, ai, aj):
        y1i, x1i, y2i, x2i = ci
        y1j, x1j, y2j, x2j = cj
        iy1 = jnp.maximum(y1i[:, :, None], y1j[:, None, :])
        ix1 = jnp.maximum(x1i[:, :, None], x1j[:, None, :])
        iy2 = jnp.minimum(y2i[:, :, None], y2j[:, None, :])
        ix2 = jnp.minimum(x2i[:, :, None], x2j[:, None, :])
        inter = jnp.maximum(iy2 - iy1, 0.0) * jnp.maximum(ix2 - ix1, 0.0)
        union = ai[:, :, None] + aj[:, None, :] - inter
        return inter / jnp.maximum(union, 1e-8)

    c_lo = (y1[:, :128], x1[:, :128], y2[:, :128], x2[:, :128])
    c_hi = (y1[:, 128:], x1[:, 128:], y2[:, 128:], x2[:, 128:])
    c_all = (y1, x1, y2, x2)
    iou0_ref[...] = _iou(c_lo, c_all, area[:, :128], area)
    iou1_ref[...] = _iou(c_hi, c_hi, area[:, 128:], area[:, 128:])

    s = s_ref[0]
    jlane = lax.broadcasted_iota(jnp.int32, (_LB, 256), 1)
    jl128 = lax.broadcasted_iota(jnp.int32, (_LB, 128), 1)

    def body0(i, keep):
        row = iou0_ref[:, pl.ds(i, 1), :].reshape(_LB, 256)
        keep_i = jnp.sum(jnp.where(jlane == i, keep, 0.0),
                         axis=1, keepdims=True)
        sup = (row > _IOU_THR) & (jlane > i) & (keep_i > 0.0)
        return jnp.where(sup, 0.0, keep)

    keep_f = lax.fori_loop(0, 128, body0,
                           jnp.where(s > _SCORE_THR, 1.0, 0.0))
    keep_lo = keep_f[:, :128]

    def body1(i2, keep_hi):
        row = iou1_ref[:, pl.ds(i2, 1), :].reshape(_LB, 128)
        keep_i = jnp.sum(jnp.where(jl128 == i2, keep_hi, 0.0),
                         axis=1, keepdims=True)
        sup = (row > _IOU_THR) & (jl128 > i2) & (keep_i > 0.0)
        return jnp.where(sup, 0.0, keep_hi)

    keep_hi = lax.fori_loop(0, 128, body1, keep_f[:, 128:])
    keep_f = jnp.concatenate([keep_lo, keep_hi], axis=1)
    keep = keep_f > 0.0
    lt = (lax.broadcasted_iota(jnp.int32, (256, 256), 0)
          <= lax.broadcasted_iota(jnp.int32, (256, 256), 1)).astype(jnp.float32)
    cum = jnp.dot(keep_f, lt, preferred_element_type=jnp.float32)
    keep = keep & (cum - 1.0 < np.float32(_MAX_TOTAL))
    out_s_ref[0] = jnp.where(keep, s, -1.0)
    out_b_ref[0, 0] = y1
    out_b_ref[0, 1] = x1
    out_b_ref[0, 2] = y2
    out_b_ref[0, 3] = x2


def _nms(s_in, rois_c, deltas_c):
    B = s_in.shape[0]
    return pl.pallas_call(
        _nms_body,
        grid=(B, _LP // _LB),
        in_specs=[
            pl.BlockSpec((1, _LB, 256), lambda b, l: (b, l, 0)),
            pl.BlockSpec((1, 4, _LB, 256), lambda b, l: (b, 0, l, 0)),
            pl.BlockSpec((1, 4, _LB, 256), lambda b, l: (b, 0, l, 0)),
        ],
        out_specs=[pl.BlockSpec((1, _LB, 256), lambda b, l: (b, l, 0)),
                   pl.BlockSpec((1, 4, _LB, 256), lambda b, l: (b, 0, l, 0))],
        out_shape=[jax.ShapeDtypeStruct((B, _LP, 256), jnp.float32),
                   jax.ShapeDtypeStruct((B, 4, _LP, 256), jnp.float32)],
        scratch_shapes=[pltpu.VMEM((_LB, 128, 256), jnp.float32),
                        pltpu.VMEM((_LB, 128, 128), jnp.float32)],
    )(s_in, rois_c, deltas_c)


def kernel(roi_bboxes, pred_deltas, pred_label_probs):
    B = roi_bboxes.shape[0]

    s1, i1 = _topk256(pred_label_probs, _N, compute_scores=True)
    s1 = jnp.transpose(s1[:, :, :_L], (0, 2, 1))
    i1 = jnp.transpose(i1[:, :, :_L], (0, 2, 1))

    s_in = jnp.concatenate(
        [s1, jnp.full((B, _LP - _L, 256), -2.0, jnp.float32)], axis=1)
    idx = jnp.concatenate(
        [i1, jnp.zeros((B, _LP - _L, 256), jnp.int32)], axis=1)
    idx = jnp.clip(idx, 0, _N - 1)

    flat_idx = idx.reshape(B, -1)
    rois_g = jax.vmap(lambda r, i: r[i])(roi_bboxes, flat_idx)
    lcls = jnp.minimum(jnp.arange(_LP, dtype=jnp.int32), _L - 1)
    cols = (4 * lcls[None, :, None, None]
            + jnp.arange(4, dtype=jnp.int32)[None, None, None, :])
    cols = jnp.broadcast_to(cols, (B, _LP, 256, 4)).reshape(B, -1)
    rows = jnp.broadcast_to(idx[..., None], (B, _LP, 256, 4)).reshape(B, -1)
    deltas_g = jax.vmap(lambda d, r, c: d[r, c])(pred_deltas, rows, cols)

    rois_c = jnp.transpose(rois_g.reshape(B, _LP, 256, 4), (0, 3, 1, 2))
    deltas_c = jnp.transpose(deltas_g.reshape(B, _LP, 256, 4), (0, 3, 1, 2))

    out_s, out_b = _nms(s_in, rois_c, deltas_c)

    s2, i2 = _gmerge(out_s.reshape(B, 192, 128))
    top_s = s2[:, :_MAX_TOTAL, 0]
    top_i = i2[:, :_MAX_TOTAL, 0]

    flat_b = jnp.transpose(out_b, (0, 2, 3, 1)).reshape(B, _LP * 256, 4)
    final_bboxes = jnp.take_along_axis(flat_b, top_i[..., None], axis=1)
    labels = (top_i // _CAND).astype(jnp.float32)
    valid = top_s > 0.0
    final_scores = jnp.where(valid, top_s, 0.0)
    final_bboxes = jnp.where(valid[..., None], final_bboxes, 0.0)
    final_labels = jnp.where(valid, labels, 0.0)
    return final_bboxes, final_labels, final_scores


# speedup vs baseline: 7.7940x; 1.0068x over previous
import functools

import numpy as np
import jax
import jax.numpy as jnp
from jax import lax
from jax.experimental import pallas as pl
from jax.experimental.pallas import tpu as pltpu

_L = 91
_LP = 96
_N = 20000
_CAND = 256
_MAX_TOTAL = 200
_SCORE_THR = 0.5
_IOU_THR = 0.5
_CHUNK = 512
_T = 64
_INT_MIN = np.int32(-2**31)
_IDX_PAD = np.int32(2**31 - 1)


def _f2key(s):
    b = lax.bitcast_convert_type(s, jnp.int32)
    return b ^ (lax.shift_right_arithmetic(b, 31) & np.int32(0x7FFFFFFF))


def _key2f(k):
    return lax.bitcast_convert_type(
        k ^ (lax.shift_right_arithmetic(k, 31) & np.int32(0x7FFFFFFF)), jnp.float32)


def _gt(ka, ia, kb, ib):
    return (ka > kb) | ((ka == kb) & (ia < ib))


def _stage(karr, iarr, T, S, j, k=None, invert=False, lane_level=None,
           dir_full=None, si=None):
    if j < T:
        G = T // (2 * j)
        k5 = karr.reshape(G, 2, j, S, 128)
        i5 = iarr.reshape(G, 2, j, S, 128)
        ka, kb = k5[:, 0], k5[:, 1]
        ia, ib = i5[:, 0], i5[:, 1]
        gtba = _gt(kb, ib, ka, ia)
        sh = (G, j, S, 128)
        if dir_full is not None:
            dir_a = dir_full.reshape(G, 2, j, S, 128)[:, 0]
        elif k is None:
            dir_a = None
        else:
            n_a = (lax.broadcasted_iota(jnp.int32, sh, 2) * T
                   + lax.broadcasted_iota(jnp.int32, sh, 0) * (2 * j)
                   + lax.broadcasted_iota(jnp.int32, sh, 1))
            dir_a = ((n_a & k) != 0) if invert else ((n_a & k) == 0)
        if lane_level is not None:
            flip = (lax.broadcasted_iota(jnp.int32, sh, 3)
                    & (1 << lane_level)) != 0
            dir_a = flip != dir_a if dir_a is not None else ~flip
        swap = gtba if dir_a is None else (gtba == dir_a)
        nak = jnp.where(swap, kb, ka)
        nbk = jnp.where(swap, ka, kb)
        nai = jnp.where(swap, ib, ia)
        nbi = jnp.where(swap, ia, ib)
        karr = jnp.concatenate([nak[:, None], nbk[:, None]], axis=1).reshape(T, S, 128)
        iarr = jnp.concatenate([nai[:, None], nbi[:, None]], axis=1).reshape(T, S, 128)
    else:
        js = j // T
        sh = (T, S, 128)
        if si is None:
            si = lax.broadcasted_iota(jnp.int32, sh, 1)
        low = (si & js) == 0
        pk = jnp.where(low, jnp.roll(karr, -js, axis=1), jnp.roll(karr, js, axis=1))
        pi = jnp.where(low, jnp.roll(iarr, -js, axis=1), jnp.roll(iarr, js, axis=1))
        gtp = _gt(pk, pi, karr, iarr)
        if dir_full is not None:
            d = dir_full
        elif k is None:
            d = None
        else:
            n = si * T + lax.broadcasted_iota(jnp.int32, sh, 0)
            d = ((n & k) != 0) if invert else ((n & k) == 0)
        if lane_level is not None:
            flip = (lax.broadcasted_iota(jnp.int32, sh, 2)
                    & (1 << lane_level)) != 0
            d = flip != d if d is not None else ~flip
        winner_here = low if d is None else (low == d)
        take = gtp == winner_here
        karr = jnp.where(take, pk, karr)
        iarr = jnp.where(take, pi, iarr)
    return karr, iarr


def _topk_body(in_ref, out_s_ref, out_i_ref, rk_ref, ri_ref, *,
               n_valid, n_chunks, compute_scores):
    c = pl.program_id(1)

    x = in_ref[0].reshape(_T, 8, 128)
    lane = lax.broadcasted_iota(jnp.int32, (_T, 8, 128), 2)
    if compute_scores:
        p = jnp.where(lane < _L, x, -1.0)
        amax = jnp.max(p, axis=2, keepdims=True)
        argl = jnp.min(jnp.where(p == amax, lane, 128), axis=2, keepdims=True)
        s = jnp.where(argl == 0, 0.0, p)
    else:
        s = x
    key = _f2key(s)
    t_i = lax.broadcasted_iota(jnp.int32, (_T, 8, 128), 0)
    s_i = lax.broadcasted_iota(jnp.int32, (_T, 8, 128), 1)
    rowid = c * _CHUNK + t_i * 8 + s_i
    key = jnp.where(rowid < n_valid, key, _INT_MIN)

    n_full = s_i * _T + t_i
    for k in [2, 4, 8, 16, 32, 64, 128, 256, 512]:
        dir_full = (n_full & k) != 0
        j = k // 2
        while j >= 1:
            key, rowid = _stage(key, rowid, _T, 8, j, k, dir_full=dir_full,
                                si=s_i)
            j //= 2

    akr, air = key[:, 4:8], rowid[:, 4:8]

    @pl.when(c == 0)
    def _init():
        rk_ref[...] = jnp.full((_T, 4, 128), _INT_MIN, jnp.int32)
        ri_ref[...] = jnp.full((_T, 4, 128), _IDX_PAD, jnp.int32)

    rk, ri = rk_ref[...], ri_ref[...]
    gta = _gt(akr, air, rk, ri)
    wk = jnp.where(gta, akr, rk)
    wi = jnp.where(gta, air, ri)
    for j in [128, 64, 32, 16, 8, 4, 2, 1]:
        wk, wi = _stage(wk, wi, _T, 4, j, None)
    rk_ref[...] = wk
    ri_ref[...] = wi

    @pl.when(c == n_chunks - 1)
    def _emit():
        sc = _key2f(wk)
        for si in range(4):
            out_s_ref[0, si] = sc[:, si, :]
            out_i_ref[0, si] = wi[:, si, :]


def _topk256(x, n_valid, compute_scores):
    B, n, _ = x.shape
    n_chunks = (n + _CHUNK - 1) // _CHUNK
    out = pl.pallas_call(
        functools.partial(_topk_body, n_valid=n_valid, n_chunks=n_chunks,
                          compute_scores=compute_scores),
        grid=(B, n_chunks),
        in_specs=[pl.BlockSpec((1, _CHUNK, 128), lambda b, c: (b, c, 0))],
        out_specs=[pl.BlockSpec((1, 4, _T, 128), lambda b, c: (b, 0, 0, 0)),
                   pl.BlockSpec((1, 4, _T, 128), lambda b, c: (b, 0, 0, 0))],
        out_shape=[jax.ShapeDtypeStruct((B, 4, _T, 128), jnp.float32),
                   jax.ShapeDtypeStruct((B, 4, _T, 128), jnp.int32)],
        scratch_shapes=[pltpu.VMEM((_T, 4, 128), jnp.int32),
                        pltpu.VMEM((_T, 4, 128), jnp.int32)],
    )(x)
    return out[0].reshape(B, 256, 128), out[1].reshape(B, 256, 128)


def _gmerge_body(in_ref, out_s_ref, out_i_ref):
    x = in_ref[0]
    keys = jnp.concatenate(
        [_f2key(x), jnp.full((64, 128), _INT_MIN, jnp.int32)], axis=0)
    keys = keys.reshape(32, 8, 128)
    sh = (32, 8, 128)
    t_i = lax.broadcasted_iota(jnp.int32, sh, 0)
    s_i = lax.broadcasted_iota(jnp.int32, sh, 1)
    l_i = lax.broadcasted_iota(jnp.int32, sh, 2)
    r = t_i * 8 + s_i
    idx = jnp.where(r < 192, r * 128 + l_i, _IDX_PAD)

    for k in [2, 4, 8, 16, 32, 64, 128, 256]:
        j = k // 2
        while j >= 1:
            keys, idx = _stage(keys, idx, 32, 8, j, k, lane_level=0)
            j //= 2

    for lev in range(1, 8):
        d = 1 << (lev - 1)
        low = (l_i & d) == 0
        pk = jnp.where(low, jnp.roll(keys, -d, axis=2), jnp.roll(keys, d, axis=2))
        pi = jnp.where(low, jnp.roll(idx, -d, axis=2), jnp.roll(idx, d, axis=2))
        win = _gt(pk, pi, keys, idx)
        keys = jnp.where(win, pk, keys)
        idx = jnp.where(win, pi, idx)
        for j in [128, 64, 32, 16, 8, 4, 2, 1]:
            keys, idx = _stage(keys, idx, 32, 8, j, None, lane_level=lev)

    sc = _key2f(keys)
    for si in range(8):
        out_s_ref[0, si] = sc[:, si, :]
        out_i_ref[0, si] = idx[:, si, :]


def _gmerge(x):
    B = x.shape[0]
    out = pl.pallas_call(
        _gmerge_body,
        grid=(B,),
        in_specs=[pl.BlockSpec((1, 192, 128), lambda b: (b, 0, 0))],
        out_specs=[pl.BlockSpec((1, 8, 32, 128), lambda b: (b, 0, 0, 0)),
                   pl.BlockSpec((1, 8, 32, 128), lambda b: (b, 0, 0, 0))],
        out_shape=[jax.ShapeDtypeStruct((B, 8, 32, 128), jnp.float32),
                   jax.ShapeDtypeStruct((B, 8, 32, 128), jnp.int32)],
    )(x)
    return out[0].reshape(B, 256, 128), out[1].reshape(B, 256, 128)


_LB = 48


def _decode(r, d):
    h = r[2] - r[0]
    w = r[3] - r[1]
    cy = r[0] + 0.5 * h
    cx = r[1] + 0.5 * w
    dy = d[0] * np.float32(0.1)
    dx = d[1] * np.float32(0.1)
    dh = d[2] * np.float32(0.2)
    dw = d[3] * np.float32(0.2)
    ph = jnp.exp(dh) * h
    pw = jnp.exp(dw) * w
    pcy = dy * h + cy
    pcx = dx * w + cx
    y1 = pcy - 0.5 * ph
    x1 = pcx - 0.5 * pw
    y2 = y1 + ph
    x2 = x1 + pw
    return y1, x1, y2, x2


def _nms_body(s_ref, r_ref, d_ref, out_s_ref, out_b_ref, iou0_ref, iou1_ref):
    y1, x1, y2, x2 = _decode(r_ref[0], d_ref[0])
    area = jnp.maximum(y2 - y1, 0.0) * jnp.maximum(x2 - x1, 0.0)

    def _iou(ci, cj, ai, aj):
        y1i, x1i, y2i, x2i = ci
        y1j, x1j, y2j, x2j = cj
        iy1 = jnp.maximum(y1i[:, :, None], y1j[:, None, :])
        ix1 = jnp.maximum(x1i[:, :, None], x1j[:, None, :])
        iy2 = jnp.minimum(y2i[:, :, None], y2j[:, None, :])
        ix2 = jnp.minimum(x2i[:, :, None], x2j[:, None, :])
        inter = jnp.maximum(iy2 - iy1, 0.0) * jnp.maximum(ix2 - ix1, 0.0)
        union = ai[:, :, None] + aj[:, None, :] - inter
        return inter / jnp.maximum(union, 1e-8)

    c_lo = (y1[:, :128], x1[:, :128], y2[:, :128], x2[:, :128])
    c_hi = (y1[:, 128:], x1[:, 128:], y2[:, 128:], x2[:, 128:])
    c_all = (y1, x1, y2, x2)
    iou0_ref[...] = _iou(c_lo, c_all, area[:, :128], area)
    iou1_ref[...] = _iou(c_hi, c_hi, area[:, 128:], area[:, 128:])

    s = s_ref[0]
    jlane = lax.broadcasted_iota(jnp.int32, (_LB, 256), 1)
    jl128 = lax.broadcasted_iota(jnp.int32, (_LB, 128), 1)

    def body0(i, keep):
        row = iou0_ref[:, pl.ds(i, 1), :].reshape(_LB, 256)
        keep_i = jnp.sum(jnp.where(jlane == i, keep, 0.0),
                         axis=1, keepdims=True)
        sup = (row > _IOU_THR) & (jlane > i) & (keep_i > 0.0)
        return jnp.where(sup, 0.0, keep)

    keep_f = lax.fori_loop(0, 128, body0,
                           jnp.where(s > _SCORE_THR, 1.0, 0.0))
    keep_lo = keep_f[:, :128]

    def body1(i2, keep_hi):
        row = iou1_ref[:, pl.ds(i2, 1), :].reshape(_LB, 128)
        keep_i = jnp.sum(jnp.where(jl128 == i2, keep_hi, 0.0),
                         axis=1, keepdims=True)
        sup = (row > _IOU_THR) & (jl128 > i2) & (keep_i > 0.0)
        return jnp.where(sup, 0.0, keep_hi)

    keep_hi = lax.fori_loop(0, 128, body1, keep_f[:, 128:])
    keep_f = jnp.concatenate([keep_lo, keep_hi], axis=1)
    keep = keep_f > 0.0
    lt = (lax.broadcasted_iota(jnp.int32, (256, 256), 0)
          <= lax.broadcasted_iota(jnp.int32, (256, 256), 1)).astype(jnp.float32)
    cum = jnp.dot(keep_f, lt, preferred_element_type=jnp.float32)
    keep = keep & (cum - 1.0 < np.float32(_MAX_TOTAL))
    out_s_ref[0] = jnp.where(keep, s, -1.0)
    out_b_ref[0, 0] = y1
    out_b_ref[0, 1] = x1
    out_b_ref[0, 2] = y2
    out_b_ref[0, 3] = x2


def _nms(s_in, rois_c, deltas_c):
    B = s_in.shape[0]
    return pl.pallas_call(
        _nms_body,
        grid=(B, _LP // _LB),
        in_specs=[
            pl.BlockSpec((1, _LB, 256), lambda b, l: (b, l, 0)),
            pl.BlockSpec((1, 4, _LB, 256), lambda b, l: (b, 0, l, 0)),
            pl.BlockSpec((1, 4, _LB, 256), lambda b, l: (b, 0, l, 0)),
        ],
        out_specs=[pl.BlockSpec((1, _LB, 256), lambda b, l: (b, l, 0)),
                   pl.BlockSpec((1, 4, _LB, 256), lambda b, l: (b, 0, l, 0))],
        out_shape=[jax.ShapeDtypeStruct((B, _LP, 256), jnp.float32),
                   jax.ShapeDtypeStruct((B, 4, _LP, 256), jnp.float32)],
        scratch_shapes=[pltpu.VMEM((_LB, 128, 256), jnp.float32),
                        pltpu.VMEM((_LB, 128, 128), jnp.float32)],
    )(s_in, rois_c, deltas_c)


def kernel(roi_bboxes, pred_deltas, pred_label_probs):
    B = roi_bboxes.shape[0]

    s1, i1 = _topk256(pred_label_probs, _N, compute_scores=True)
    s1 = jnp.transpose(s1[:, :, :_L], (0, 2, 1))
    i1 = jnp.transpose(i1[:, :, :_L], (0, 2, 1))

    s_in = jnp.concatenate(
        [s1, jnp.full((B, _LP - _L, 256), -2.0, jnp.float32)], axis=1)
    idx = jnp.concatenate(
        [i1, jnp.zeros((B, _LP - _L, 256), jnp.int32)], axis=1)
    idx = jnp.clip(idx, 0, _N - 1)

    flat_idx = idx.reshape(B, -1)
    rois_g = jax.vmap(lambda r, i: r[i])(roi_bboxes, flat_idx)
    lcls = jnp.minimum(jnp.arange(_LP, dtype=jnp.int32), _L - 1)
    cols = (4 * lcls[None, :, None, None]
            + jnp.arange(4, dtype=jnp.int32)[None, None, None, :])
    cols = jnp.broadcast_to(cols, (B, _LP, 256, 4)).reshape(B, -1)
    rows = jnp.broadcast_to(idx[..., None], (B, _LP, 256, 4)).reshape(B, -1)
    deltas_g = jax.vmap(lambda d, r, c: d[r, c])(pred_deltas, rows, cols)

    rois_c = jnp.transpose(rois_g.reshape(B, _LP, 256, 4), (0, 3, 1, 2))
    deltas_c = jnp.transpose(deltas_g.reshape(B, _LP, 256, 4), (0, 3, 1, 2))

    out_s, out_b = _nms(s_in, rois_c, deltas_c)

    s2, i2 = _gmerge(out_s.reshape(B, 192, 128))
    top_s = s2[:, :_MAX_TOTAL, 0]
    top_i = i2[:, :_MAX_TOTAL, 0]

    flat_b = jnp.transpose(out_b, (0, 2, 3, 1)).reshape(B, _LP * 256, 4)
    final_bboxes = jnp.take_along_axis(flat_b, top_i[..., None], axis=1)
    labels = (top_i // _CAND).astype(jnp.float32)
    valid = top_s > 0.0
    final_scores = jnp.where(valid, top_s, 0.0)
    final_bboxes = jnp.where(valid[..., None], final_bboxes, 0.0)
    final_labels = jnp.where(valid, labels, 0.0)
    return final_bboxes, final_labels, final_scores
